# Initial kernel scaffold; baseline (speedup 1.0000x reference)
import jax
import jax.numpy as jnp
from jax.experimental import pallas as pl


def kernel(x, edge_index, params):
    raise NotImplementedError("write your pallas kernel here")


# speedup vs baseline: 3.2779x; 3.2779x over previous
import functools

import jax
import jax.numpy as jnp
from jax import lax
from jax.experimental import pallas as pl
from jax.experimental.pallas import tpu as pltpu
from jax.experimental.pallas import tpu_sc as plsc

_N = 10000
_D = 128
_NC = 2
_NS = 16
_NW = _NC * _NS
_C = 128
_E = 320000
_G = -(-_E // (_NW * _C))
_EPAD = _NW * _G * _C
_NROWS = _N + 112
_RPB = _NROWS // _NS

_mesh = plsc.VectorSubcoreMesh(core_axis_name="c", subcore_axis_name="s")


@functools.partial(
    pl.kernel, mesh=_mesh,
    out_type=jax.ShapeDtypeStruct((_NC * _NROWS, _D), jnp.float32),
    scratch_types=[
        pltpu.VMEM((_C,), jnp.int32),
        pltpu.VMEM((_C,), jnp.int32),
        pltpu.VMEM((_C, _D), jnp.float32),
        pltpu.VMEM_SHARED((_NROWS, _D), jnp.float32),
        pltpu.SemaphoreType.DMA,
    ])
def _agg(table, srcp, dstp, out, src_v, dst_v, rows_v, shacc, sem):
    cid = lax.axis_index("c")
    sid = lax.axis_index("s")
    wid = sid * _NC + cid
    r0 = sid * _RPB

    def zrow(r, _):
        for c8 in range(_D // 16):
            rows_v[r, pl.ds(c8 * 16, 16)] = jnp.zeros((16,), jnp.float32)
        return _
    lax.fori_loop(0, _C, zrow, None)
    nz = -(-_RPB // _C)

    def zcopy(i, _):
        off = jnp.minimum(r0 + i * _C, r0 + _RPB - _C)
        pltpu.sync_copy(rows_v, shacc.at[pl.ds(off, _C)])
        return _
    lax.fori_loop(0, nz, zcopy, None)

    plsc.subcore_barrier()

    base = wid * (_G * _C)

    def chunk(g, _):
        off = base + g * _C
        pltpu.sync_copy(srcp.at[pl.ds(off, _C)], src_v)
        pltpu.sync_copy(dstp.at[pl.ds(off, _C)], dst_v)
        pltpu.async_copy(table.at[src_v], rows_v, sem).wait()
        pltpu.sync_copy(rows_v, shacc.at[dst_v], add=True)
        return _
    lax.fori_loop(0, _G, chunk, None)

    plsc.subcore_barrier()
    pltpu.sync_copy(shacc.at[pl.ds(r0, _RPB)],
                    out.at[pl.ds(cid * _NROWS + r0, _RPB)])


@functools.partial(
    pl.kernel, mesh=_mesh,
    out_type=jax.ShapeDtypeStruct((_NC * _NROWS, _D), jnp.float32),
    scratch_types=[
        pltpu.VMEM((_C,), jnp.int32),
        pltpu.VMEM((_C, _D), jnp.float32),
        pltpu.VMEM_SHARED((_NROWS, _D), jnp.float32),
    ])
def _count(dstp, out, dst_v, ones_v, cacc):
    cid = lax.axis_index("c")
    sid = lax.axis_index("s")
    wid = sid * _NC + cid
    r0 = sid * _RPB

    def zrow(r, _):
        for c8 in range(_D // 16):
            ones_v[r, pl.ds(c8 * 16, 16)] = jnp.zeros((16,), jnp.float32)
        return _
    lax.fori_loop(0, _C, zrow, None)
    nz = -(-_RPB // _C)

    def zcopy(i, _):
        off = jnp.minimum(r0 + i * _C, r0 + _RPB - _C)
        pltpu.sync_copy(ones_v, cacc.at[pl.ds(off, _C)])
        return _
    lax.fori_loop(0, nz, zcopy, None)

    def orow(r, _):
        for c8 in range(_D // 16):
            ones_v[r, pl.ds(c8 * 16, 16)] = jnp.ones((16,), jnp.float32)
        return _
    lax.fori_loop(0, _C, orow, None)

    plsc.subcore_barrier()

    base = wid * (_G * _C)

    def chunk(g, _):
        off = base + g * _C
        pltpu.sync_copy(dstp.at[pl.ds(off, _C)], dst_v)
        pltpu.sync_copy(ones_v, cacc.at[dst_v], add=True)
        return _
    lax.fori_loop(0, _G, chunk, None)

    plsc.subcore_barrier()
    pltpu.sync_copy(cacc.at[pl.ds(r0, _RPB)],
                    out.at[pl.ds(cid * _NROWS + r0, _RPB)])


@functools.partial(
    pl.kernel, mesh=_mesh,
    out_type=jax.ShapeDtypeStruct((_EPAD, 16), jnp.float32),
    scratch_types=[
        pltpu.VMEM((_C,), jnp.int32),
        pltpu.VMEM((_C,), jnp.int32),
        pltpu.VMEM((_C, _D), jnp.float32),
        pltpu.VMEM((_C, _D), jnp.float32),
        pltpu.VMEM((_D,), jnp.float32),
        pltpu.VMEM((_C, 16), jnp.float32),
        pltpu.SemaphoreType.DMA,
        pltpu.SemaphoreType.DMA,
    ])
def _edge_mlp(hA, hB, srcp, dstp, w2, out, src_v, dst_v, rowsA, rowsB,
              w2_v, pbuf, semA, semB):
    cid = lax.axis_index("c")
    sid = lax.axis_index("s")
    wid = sid * _NC + cid
    base = wid * (_G * _C)
    pltpu.sync_copy(w2, w2_v)

    def chunk(g, _):
        off = base + g * _C
        pltpu.sync_copy(srcp.at[pl.ds(off, _C)], src_v)
        pltpu.sync_copy(dstp.at[pl.ds(off, _C)], dst_v)
        ca = pltpu.async_copy(hA.at[src_v], rowsA, semA)
        cb = pltpu.async_copy(hB.at[dst_v], rowsB, semB)
        ca.wait()
        cb.wait()

        def edge(e, _):
            acc = jnp.zeros((16,), jnp.float32)
            for k in range(_D // 16):
                va = rowsA[e, pl.ds(k * 16, 16)]
                vb = rowsB[e, pl.ds(k * 16, 16)]
                w = w2_v[pl.ds(k * 16, 16)]
                acc = acc + jnp.maximum(va + vb, 0.0) * w
            pbuf[e, pl.ds(0, 16)] = acc
            return _
        lax.fori_loop(0, _C, edge, None)

        pltpu.sync_copy(pbuf, out.at[pl.ds(off, _C)])
        return _
    lax.fori_loop(0, _G, chunk, None)


_EBN = 4096
_ENB = _EPAD // _EBN


def _efin_body(p16, be2, out):
    s = jnp.sum(p16[...], axis=1, keepdims=True) + be2[...]
    out[...] = jax.nn.sigmoid(s)


_efin = pl.pallas_call(
    _efin_body,
    grid=(_ENB,),
    in_specs=[pl.BlockSpec((_EBN, 16), lambda i: (i, 0)),
              pl.BlockSpec((1, 1), lambda i: (0, 0))],
    out_specs=pl.BlockSpec((_EBN, 1), lambda i: (i, 0)),
    out_shape=jax.ShapeDtypeStruct((_EPAD, 1), jnp.float32),
)


_BN = 1000
_NB = _N // _BN


def _rows(bn=_BN, d=_D):
    return pl.BlockSpec((bn, d), lambda i: (i, 0))


def _wmat():
    return pl.BlockSpec((_D, _D), lambda i: (0, 0))


def _brow():
    return pl.BlockSpec((1, _D), lambda i: (0, 0))


def _f32(*shape):
    return jax.ShapeDtypeStruct(shape, jnp.float32)


def _dot(a, b):
    return jnp.dot(a, b, preferred_element_type=jnp.float32)


def _tpre_body(x, p0, p1, c0, c1, WLr, WRr, br, WLz, WRz, bz, WLc, WRc, bc,
               bsh, bb1, bb2, WB2,
               inv_o, rx_o, zx_o, cx_o, hn1_o, ahn1_o):
    cnt = c0[:, 0:1] + c1[:, 0:1]
    inv = 1.0 / jnp.maximum(cnt, 1.0)
    invb = jnp.broadcast_to(inv, (_BN, _D))
    inv_o[...] = invb
    A = (p0[...] + p1[...]) * invb
    xx = x[...]
    rx_o[...] = _dot(A, WLr[...]) + _dot(xx, WRr[...]) + br[...]
    zx_o[...] = _dot(A, WLz[...]) + _dot(xx, WRz[...]) + bz[...]
    cx_o[...] = _dot(A, WLc[...]) + _dot(xx, WRc[...]) + bc[...]
    beta = jnp.tanh(bb1[...] + _dot(bsh[...], WB2[...]) + bb2[...])
    hn1_o[...] = jnp.broadcast_to(beta, (_BN, _D))
    ahn1_o[...] = jnp.where(cnt > 0.0, 1.0, 0.0) * beta


_tpre = pl.pallas_call(
    _tpre_body,
    grid=(_NB,),
    in_specs=[_rows(), _rows(), _rows(), _rows(), _rows(),
              _wmat(), _wmat(), _brow(), _wmat(), _wmat(), _brow(),
              _wmat(), _wmat(), _brow(), _brow(), _brow(), _brow(), _wmat()],
    out_specs=[_rows()] * 6,
    out_shape=[_f32(_N, _D)] * 6,
)


def _t1_body(h, p0, p1, inv, WLs, WRs, bs, WB1, bb1, WB2, bb2, hn_o):
    A = (p0[...] + p1[...]) * inv[...]
    hh = h[...]
    hN0 = _dot(A, WLs[...]) + _dot(hh, WRs[...]) + bs[...]
    beta = jnp.tanh(_dot(hh, WB1[...]) + bb1[...] + _dot(hN0, WB2[...]) + bb2[...])
    hn_o[...] = hh + beta


_t1 = pl.pallas_call(
    _t1_body,
    grid=(_NB,),
    in_specs=[_rows()] * 4 + [_wmat(), _wmat(), _brow(), _wmat(), _brow(),
                              _wmat(), _brow()],
    out_specs=_rows(),
    out_shape=_f32(_N, _D),
)


def _t2_body(combine, *args):
    if combine:
        (rx, zx, hN, p0, p1, inv, WLr, WRr, br, WLz, WRz, bz, q_o, z_o) = args
        A = (p0[...] + p1[...]) * inv[...]
    else:
        (rx, zx, hN, Ai, WLr, WRr, br, WLz, WRz, bz, q_o, z_o) = args
        A = Ai[...]
    h = hN[...]
    r = jax.nn.sigmoid(rx[...] + _dot(A, WLr[...]) + _dot(h, WRr[...]) + br[...])
    z = jax.nn.sigmoid(zx[...] + _dot(A, WLz[...]) + _dot(h, WRz[...]) + bz[...])
    q_o[...] = r * h
    z_o[...] = z


_t2_direct = pl.pallas_call(
    functools.partial(_t2_body, False),
    grid=(_NB,),
    in_specs=[_rows()] * 4 + [_wmat(), _wmat(), _brow(), _wmat(), _wmat(),
                              _brow()],
    out_specs=[_rows()] * 2,
    out_shape=[_f32(_N, _D)] * 2,
)

_t2_comb = pl.pallas_call(
    functools.partial(_t2_body, True),
    grid=(_NB,),
    in_specs=[_rows()] * 6 + [_wmat(), _wmat(), _brow(), _wmat(), _wmat(),
                              _brow()],
    out_specs=[_rows()] * 2,
    out_shape=[_f32(_N, _D)] * 2,
)


def _t3_body(final, *args):
    if final:
        (cx, q, z, hN, p0, p1, inv, WLc, WRc, bc, WA, ba, WB,
         h_o, ha_o, hb_o) = args
    else:
        (cx, q, z, hN, p0, p1, inv, WLc, WRc, bc, h_o) = args
    A = (p0[...] + p1[...]) * inv[...]
    qq = q[...]
    ht = jnp.tanh(cx[...] + _dot(A, WLc[...]) + _dot(qq, WRc[...]) + bc[...])
    zz = z[...]
    h = (1.0 - zz) * hN[...] + zz * ht
    h_o[...] = h
    if final:
        ha_o[...] = _dot(h, WA[...]) + ba[...]
        hb_o[...] = _dot(h, WB[...])


_t3 = pl.pallas_call(
    functools.partial(_t3_body, False),
    grid=(_NB,),
    in_specs=[_rows()] * 7 + [_wmat(), _wmat(), _brow()],
    out_specs=_rows(),
    out_shape=_f32(_N, _D),
)

_t3f = pl.pallas_call(
    functools.partial(_t3_body, True),
    grid=(_NB,),
    in_specs=[_rows()] * 7 + [_wmat(), _wmat(), _brow(), _wmat(), _brow(),
                              _wmat()],
    out_specs=[_rows()] * 3,
    out_shape=[_f32(_N, _D)] * 3,
)


def _slices(part):
    return part[0:_N], part[_NROWS:_NROWS + _N]


def kernel(x, edge_index, params):
    p = params
    src = edge_index[0]
    dst = edge_index[1]
    npad = _EPAD - _E
    pad0 = jnp.zeros((npad,), jnp.int32)
    src_p = jnp.concatenate([src, pad0])
    dst_agg = jnp.concatenate([dst, jnp.full((npad,), _N, jnp.int32)])
    dst_edge = jnp.concatenate([dst, pad0])

    def b(v):
        return jnp.reshape(v, (1, _D))

    cntp = _count(dst_agg)
    axp = _agg(x, src_p, dst_agg)
    ax0, ax1 = _slices(axp)
    c0, c1 = _slices(cntp)

    inv, Rx, Zx, Cx, hN1, AhN1 = _tpre(
        x, ax0, ax1, c0, c1,
        p["ssx"]["Wl"].T, p["ssx"]["Wr"].T, b(p["ssx"]["b"]),
        p["sux"]["Wl"].T, p["sux"]["Wr"].T, b(p["sux"]["b"]),
        p["scx"]["Wl"].T, p["scx"]["Wr"].T, b(p["scx"]["b"]),
        b(p["sh"]["b"]), b(p["bb1"]), b(p["bb2"]), p["Wb2"].T)

    q1, z1 = _t2_direct(
        Rx, Zx, hN1, AhN1,
        p["ssh"]["Wl"].T, p["ssh"]["Wr"].T, b(p["ssh"]["b"]),
        p["suh"]["Wl"].T, p["suh"]["Wr"].T, b(p["suh"]["b"]))
    aq1p = _agg(q1, src_p, dst_agg)
    aq10, aq11 = _slices(aq1p)
    h1 = _t3(Cx, q1, z1, hN1, aq10, aq11, inv,
             p["sch"]["Wl"].T, p["sch"]["Wr"].T, b(p["sch"]["b"]))

    ahp = _agg(h1, src_p, dst_agg)
    ah0, ah1 = _slices(ahp)
    hN2 = _t1(h1, ah0, ah1, inv,
              p["sh"]["Wl"].T, p["sh"]["Wr"].T, b(p["sh"]["b"]),
              p["Wb1"].T, b(p["bb1"]), p["Wb2"].T, b(p["bb2"]))
    ahnp = _agg(hN2, src_p, dst_agg)
    ahn0, ahn1 = _slices(ahnp)
    q2, z2 = _t2_comb(
        Rx, Zx, hN2, ahn0, ahn1, inv,
        p["ssh"]["Wl"].T, p["ssh"]["Wr"].T, b(p["ssh"]["b"]),
        p["suh"]["Wl"].T, p["suh"]["Wr"].T, b(p["suh"]["b"]))
    aq2p = _agg(q2, src_p, dst_agg)
    aq20, aq21 = _slices(aq2p)
    h2, hA, hB = _t3f(Cx, q2, z2, hN2, aq20, aq21, inv,
                      p["sch"]["Wl"].T, p["sch"]["Wr"].T, b(p["sch"]["b"]),
                      p["We1"][:, :_D].T, b(p["be1"]), p["We1"][:, _D:].T)

    p16 = _edge_mlp(hA, hB, src_p, dst_edge, p["We2"][0])
    pred_pad = _efin(p16, jnp.reshape(p["be2"], (1, 1)))
    pred = pred_pad[:_E]
    return (pred, h2)


# speedup vs baseline: 4.0314x; 1.2299x over previous
import functools

import jax
import jax.numpy as jnp
from jax import lax
from jax.experimental import pallas as pl
from jax.experimental.pallas import tpu as pltpu
from jax.experimental.pallas import tpu_sc as plsc

_N = 10000
_D = 128
_NC = 2
_NS = 16
_NW = _NC * _NS
_E = 320000


def _chunking(c):
    g = -(-_E // (_NW * c))
    g += g % 2
    return g, _NW * g * c


_CA = 64
_GA, _EPA = _chunking(_CA)
_C = 128
_G, _EPAD = _chunking(_C)
_NROWS = _N + 112
_RPB = _NROWS // _NS

_mesh = plsc.VectorSubcoreMesh(core_axis_name="c", subcore_axis_name="s")


@functools.partial(
    pl.kernel, mesh=_mesh,
    out_type=jax.ShapeDtypeStruct((_NC * _NROWS, _D), jnp.float32),
    scratch_types=[
        pltpu.VMEM((_GA * _CA,), jnp.int32),
        pltpu.VMEM((_CA,), jnp.int32),
        pltpu.VMEM((_CA,), jnp.int32),
        pltpu.VMEM((_CA, _D), jnp.float32),
        pltpu.VMEM((_CA, _D), jnp.float32),
        pltpu.VMEM_SHARED((_NROWS, _D), jnp.float32),
        pltpu.SemaphoreType.DMA,
        pltpu.SemaphoreType.DMA,
        pltpu.SemaphoreType.DMA,
        pltpu.SemaphoreType.DMA,
    ])
def _agg(table, srcp, dstp, out, src_all, dst_v0, dst_v1, rows0, rows1,
         shacc, semG0, semG1, semD0, semD1):
    cid = lax.axis_index("c")
    sid = lax.axis_index("s")
    wid = sid * _NC + cid
    r0 = sid * _RPB
    base = wid * (_GA * _CA)

    pltpu.sync_copy(srcp.at[pl.ds(base, _GA * _CA)], src_all)

    def zrow(r, _):
        for c8 in range(_D // 16):
            rows0[r, pl.ds(c8 * 16, 16)] = jnp.zeros((16,), jnp.float32)
        return _
    lax.fori_loop(0, _CA, zrow, None)
    nz = -(-_RPB // _CA)

    def zcopy(i, _):
        off = jnp.minimum(r0 + i * _CA, r0 + _RPB - _CA)
        pltpu.sync_copy(rows0, shacc.at[pl.ds(off, _CA)])
        return _
    lax.fori_loop(0, nz, zcopy, None)

    def gather(g, rows, sem):
        return pltpu.make_async_copy(
            table.at[src_all.at[pl.ds(g * _CA, _CA)]], rows, sem)

    def dload(g, dst_v, sem):
        return pltpu.make_async_copy(
            dstp.at[pl.ds(base + g * _CA, _CA)], dst_v, sem)

    pltpu.async_copy(dstp.at[pl.ds(base, _CA)], dst_v0, semD0)
    pltpu.async_copy(table.at[src_all.at[pl.ds(0, _CA)]], rows0, semG0)
    plsc.subcore_barrier()

    def outer(g2, _):
        g0 = 2 * g2
        dload(g0 + 1, dst_v1, semD1).start()
        gather(g0 + 1, rows1, semG1).start()
        gather(g0, rows0, semG0).wait()
        dload(g0, dst_v0, semD0).wait()
        pltpu.sync_copy(rows0, shacc.at[dst_v0], add=True)

        @pl.when(g2 + 1 < _GA // 2)
        def _pref():
            dload(g0 + 2, dst_v0, semD0).start()
            gather(g0 + 2, rows0, semG0).start()
        gather(g0 + 1, rows1, semG1).wait()
        dload(g0 + 1, dst_v1, semD1).wait()
        pltpu.sync_copy(rows1, shacc.at[dst_v1], add=True)
        return _
    lax.fori_loop(0, _GA // 2, outer, None)

    plsc.subcore_barrier()
    pltpu.sync_copy(shacc.at[pl.ds(r0, _RPB)],
                    out.at[pl.ds(cid * _NROWS + r0, _RPB)])


@functools.partial(
    pl.kernel, mesh=_mesh,
    out_type=jax.ShapeDtypeStruct((_NC * _NROWS, _D), jnp.float32),
    scratch_types=[
        pltpu.VMEM((_CA,), jnp.int32),
        pltpu.VMEM((_CA,), jnp.int32),
        pltpu.VMEM((_CA, _D), jnp.float32),
        pltpu.VMEM_SHARED((_NROWS, _D), jnp.float32),
        pltpu.SemaphoreType.DMA,
        pltpu.SemaphoreType.DMA,
    ])
def _count(dstp, out, dst_v0, dst_v1, ones_v, cacc, semD0, semD1):
    cid = lax.axis_index("c")
    sid = lax.axis_index("s")
    wid = sid * _NC + cid
    r0 = sid * _RPB
    base = wid * (_GA * _CA)

    def zrow(r, _):
        for c8 in range(_D // 16):
            ones_v[r, pl.ds(c8 * 16, 16)] = jnp.zeros((16,), jnp.float32)
        return _
    lax.fori_loop(0, _CA, zrow, None)
    nz = -(-_RPB // _CA)

    def zcopy(i, _):
        off = jnp.minimum(r0 + i * _CA, r0 + _RPB - _CA)
        pltpu.sync_copy(ones_v, cacc.at[pl.ds(off, _CA)])
        return _
    lax.fori_loop(0, nz, zcopy, None)

    def orow(r, _):
        for c8 in range(_D // 16):
            ones_v[r, pl.ds(c8 * 16, 16)] = jnp.ones((16,), jnp.float32)
        return _
    lax.fori_loop(0, _CA, orow, None)

    def dload(g, dst_v, sem):
        return pltpu.make_async_copy(
            dstp.at[pl.ds(base + g * _CA, _CA)], dst_v, sem)

    pltpu.async_copy(dstp.at[pl.ds(base, _CA)], dst_v0, semD0)
    plsc.subcore_barrier()

    def outer(g2, _):
        g0 = 2 * g2
        dload(g0 + 1, dst_v1, semD1).start()
        dload(g0, dst_v0, semD0).wait()
        pltpu.sync_copy(ones_v, cacc.at[dst_v0], add=True)

        @pl.when(g2 + 1 < _GA // 2)
        def _pref():
            dload(g0 + 2, dst_v0, semD0).start()
        dload(g0 + 1, dst_v1, semD1).wait()
        pltpu.sync_copy(ones_v, cacc.at[dst_v1], add=True)
        return _
    lax.fori_loop(0, _GA // 2, outer, None)

    plsc.subcore_barrier()
    pltpu.sync_copy(cacc.at[pl.ds(r0, _RPB)],
                    out.at[pl.ds(cid * _NROWS + r0, _RPB)])


@functools.partial(
    pl.kernel, mesh=_mesh,
    out_type=jax.ShapeDtypeStruct((_EPAD, 16), jnp.float32),
    scratch_types=[
        pltpu.VMEM((_G, _C), jnp.int32),
        pltpu.VMEM((_G, _C), jnp.int32),
        pltpu.VMEM((_C, _D), jnp.float32),
        pltpu.VMEM((_C, _D), jnp.float32),
        pltpu.VMEM((_C, _D), jnp.float32),
        pltpu.VMEM((_C, _D), jnp.float32),
        pltpu.VMEM((_D,), jnp.float32),
        pltpu.VMEM((_C, 16), jnp.float32),
        pltpu.VMEM((_C, 16), jnp.float32),
        pltpu.SemaphoreType.DMA,
        pltpu.SemaphoreType.DMA,
        pltpu.SemaphoreType.DMA,
        pltpu.SemaphoreType.DMA,
        pltpu.SemaphoreType.DMA,
        pltpu.SemaphoreType.DMA,
    ])
def _edge_mlp(hA, hB, srcp3, dstp3, w2, out, src_all, dst_all,
              rowsA0, rowsB0, rowsA1, rowsB1, w2_v, pbuf0, pbuf1,
              semA0, semB0, semA1, semB1, semO0, semO1):
    cid = lax.axis_index("c")
    sid = lax.axis_index("s")
    wid = sid * _NC + cid
    base = wid * (_G * _C)

    pltpu.sync_copy(srcp3.at[wid], src_all)
    pltpu.sync_copy(dstp3.at[wid], dst_all)
    pltpu.sync_copy(w2, w2_v)
    wregs = [w2_v[pl.ds(k * 16, 16)] for k in range(_D // 16)]

    pltpu.async_copy(hA.at[src_all.at[0]], rowsA0, semA0)
    pltpu.async_copy(hB.at[dst_all.at[0]], rowsB0, semB0)

    def compute(rowsA, rowsB, pbuf):
        def edge(e, _):
            acc = jnp.zeros((16,), jnp.float32)
            for k in range(_D // 16):
                va = rowsA[e, pl.ds(k * 16, 16)]
                vb = rowsB[e, pl.ds(k * 16, 16)]
                acc = acc + jnp.maximum(va + vb, 0.0) * wregs[k]
            pbuf[e, pl.ds(0, 16)] = acc
            return _
        lax.fori_loop(0, _C, edge, None)

    def outer(g2, _):
        g0 = 2 * g2
        pltpu.async_copy(hA.at[src_all.at[g0 + 1]], rowsA1, semA1)
        pltpu.async_copy(hB.at[dst_all.at[g0 + 1]], rowsB1, semB1)
        pltpu.make_async_copy(hA.at[src_all.at[g0]], rowsA0, semA0).wait()
        pltpu.make_async_copy(hB.at[dst_all.at[g0]], rowsB0, semB0).wait()

        @pl.when(g2 > 0)
        def _drain0():
            pltpu.make_async_copy(
                pbuf0, out.at[pl.ds(base + (g0 - 2) * _C, _C)], semO0).wait()
        compute(rowsA0, rowsB0, pbuf0)
        pltpu.async_copy(pbuf0, out.at[pl.ds(base + g0 * _C, _C)], semO0)

        @pl.when(g2 + 1 < _G // 2)
        def _pref():
            pltpu.async_copy(hA.at[src_all.at[g0 + 2]], rowsA0, semA0)
            pltpu.async_copy(hB.at[dst_all.at[g0 + 2]], rowsB0, semB0)
        pltpu.make_async_copy(hA.at[src_all.at[g0 + 1]], rowsA1, semA1).wait()
        pltpu.make_async_copy(hB.at[dst_all.at[g0 + 1]], rowsB1, semB1).wait()

        @pl.when(g2 > 0)
        def _drain1():
            pltpu.make_async_copy(
                pbuf1, out.at[pl.ds(base + (g0 - 1) * _C, _C)], semO1).wait()
        compute(rowsA1, rowsB1, pbuf1)
        pltpu.async_copy(pbuf1, out.at[pl.ds(base + (g0 + 1) * _C, _C)], semO1)
        return _
    lax.fori_loop(0, _G // 2, outer, None)

    pltpu.make_async_copy(pbuf0, out.at[pl.ds(base + (_G - 2) * _C, _C)],
                          semO0).wait()
    pltpu.make_async_copy(pbuf1, out.at[pl.ds(base + (_G - 1) * _C, _C)],
                          semO1).wait()


_EBN = 4096
_ENB = _EPAD // _EBN


def _efin_body(p16, be2, out):
    s = jnp.sum(p16[...], axis=1, keepdims=True) + be2[...]
    out[...] = jax.nn.sigmoid(s)


_efin = pl.pallas_call(
    _efin_body,
    grid=(_ENB,),
    in_specs=[pl.BlockSpec((_EBN, 16), lambda i: (i, 0)),
              pl.BlockSpec((1, 1), lambda i: (0, 0))],
    out_specs=pl.BlockSpec((_EBN, 1), lambda i: (i, 0)),
    out_shape=jax.ShapeDtypeStruct((_EPAD, 1), jnp.float32),
)


_BN = 1000
_NB = _N // _BN


def _rows(bn=_BN, d=_D):
    return pl.BlockSpec((bn, d), lambda i: (i, 0))


def _wmat():
    return pl.BlockSpec((_D, _D), lambda i: (0, 0))


def _brow():
    return pl.BlockSpec((1, _D), lambda i: (0, 0))


def _f32(*shape):
    return jax.ShapeDtypeStruct(shape, jnp.float32)


def _dot(a, b):
    return jnp.dot(a, b, preferred_element_type=jnp.float32)


def _tpre_body(x, p0, p1, c0, c1, WLr, WRr, br, WLz, WRz, bz, WLc, WRc, bc,
               bsh, bb1, bb2, WB2,
               inv_o, rx_o, zx_o, cx_o, hn1_o, ahn1_o):
    cnt = c0[:, 0:1] + c1[:, 0:1]
    inv = 1.0 / jnp.maximum(cnt, 1.0)
    invb = jnp.broadcast_to(inv, (_BN, _D))
    inv_o[...] = invb
    A = (p0[...] + p1[...]) * invb
    xx = x[...]
    rx_o[...] = _dot(A, WLr[...]) + _dot(xx, WRr[...]) + br[...]
    zx_o[...] = _dot(A, WLz[...]) + _dot(xx, WRz[...]) + bz[...]
    cx_o[...] = _dot(A, WLc[...]) + _dot(xx, WRc[...]) + bc[...]
    beta = jnp.tanh(bb1[...] + _dot(bsh[...], WB2[...]) + bb2[...])
    hn1_o[...] = jnp.broadcast_to(beta, (_BN, _D))
    ahn1_o[...] = jnp.where(cnt > 0.0, 1.0, 0.0) * beta


_tpre = pl.pallas_call(
    _tpre_body,
    grid=(_NB,),
    in_specs=[_rows(), _rows(), _rows(), _rows(), _rows(),
              _wmat(), _wmat(), _brow(), _wmat(), _wmat(), _brow(),
              _wmat(), _wmat(), _brow(), _brow(), _brow(), _brow(), _wmat()],
    out_specs=[_rows()] * 6,
    out_shape=[_f32(_N, _D)] * 6,
)


def _t1_body(h, p0, p1, inv, WLs, WRs, bs, WB1, bb1, WB2, bb2, hn_o):
    A = (p0[...] + p1[...]) * inv[...]
    hh = h[...]
    hN0 = _dot(A, WLs[...]) + _dot(hh, WRs[...]) + bs[...]
    beta = jnp.tanh(_dot(hh, WB1[...]) + bb1[...] + _dot(hN0, WB2[...]) + bb2[...])
    hn_o[...] = hh + beta


_t1 = pl.pallas_call(
    _t1_body,
    grid=(_NB,),
    in_specs=[_rows()] * 4 + [_wmat(), _wmat(), _brow(), _wmat(), _brow(),
                              _wmat(), _brow()],
    out_specs=_rows(),
    out_shape=_f32(_N, _D),
)


def _t2_body(combine, *args):
    if combine:
        (rx, zx, hN, p0, p1, inv, WLr, WRr, br, WLz, WRz, bz, q_o, z_o) = args
        A = (p0[...] + p1[...]) * inv[...]
    else:
        (rx, zx, hN, Ai, WLr, WRr, br, WLz, WRz, bz, q_o, z_o) = args
        A = Ai[...]
    h = hN[...]
    r = jax.nn.sigmoid(rx[...] + _dot(A, WLr[...]) + _dot(h, WRr[...]) + br[...])
    z = jax.nn.sigmoid(zx[...] + _dot(A, WLz[...]) + _dot(h, WRz[...]) + bz[...])
    q_o[...] = r * h
    z_o[...] = z


_t2_direct = pl.pallas_call(
    functools.partial(_t2_body, False),
    grid=(_NB,),
    in_specs=[_rows()] * 4 + [_wmat(), _wmat(), _brow(), _wmat(), _wmat(),
                              _brow()],
    out_specs=[_rows()] * 2,
    out_shape=[_f32(_N, _D)] * 2,
)

_t2_comb = pl.pallas_call(
    functools.partial(_t2_body, True),
    grid=(_NB,),
    in_specs=[_rows()] * 6 + [_wmat(), _wmat(), _brow(), _wmat(), _wmat(),
                              _brow()],
    out_specs=[_rows()] * 2,
    out_shape=[_f32(_N, _D)] * 2,
)


def _t3_body(final, *args):
    if final:
        (cx, q, z, hN, p0, p1, inv, WLc, WRc, bc, WA, ba, WB,
         h_o, ha_o, hb_o) = args
    else:
        (cx, q, z, hN, p0, p1, inv, WLc, WRc, bc, h_o) = args
    A = (p0[...] + p1[...]) * inv[...]
    qq = q[...]
    ht = jnp.tanh(cx[...] + _dot(A, WLc[...]) + _dot(qq, WRc[...]) + bc[...])
    zz = z[...]
    h = (1.0 - zz) * hN[...] + zz * ht
    h_o[...] = h
    if final:
        ha_o[...] = _dot(h, WA[...]) + ba[...]
        hb_o[...] = _dot(h, WB[...])


_t3 = pl.pallas_call(
    functools.partial(_t3_body, False),
    grid=(_NB,),
    in_specs=[_rows()] * 7 + [_wmat(), _wmat(), _brow()],
    out_specs=_rows(),
    out_shape=_f32(_N, _D),
)

_t3f = pl.pallas_call(
    functools.partial(_t3_body, True),
    grid=(_NB,),
    in_specs=[_rows()] * 7 + [_wmat(), _wmat(), _brow(), _wmat(), _brow(),
                              _wmat()],
    out_specs=[_rows()] * 3,
    out_shape=[_f32(_N, _D)] * 3,
)


def _slices(part):
    return part[0:_N], part[_NROWS:_NROWS + _N]


def kernel(x, edge_index, params):
    p = params
    src = edge_index[0]
    dst = edge_index[1]
    pad_a = jnp.zeros((_EPA - _E,), jnp.int32)
    pad_e = jnp.zeros((_EPAD - _E,), jnp.int32)
    src_p = jnp.concatenate([src, pad_a])
    dst_agg = jnp.concatenate([dst, jnp.full((_EPA - _E,), _N, jnp.int32)])
    src_edge = jnp.concatenate([src, pad_e]).reshape(_NW, _G, _C)
    dst_edge = jnp.concatenate([dst, pad_e]).reshape(_NW, _G, _C)

    def b(v):
        return jnp.reshape(v, (1, _D))

    cntp = _count(dst_agg)
    axp = _agg(x, src_p, dst_agg)
    ax0, ax1 = _slices(axp)
    c0, c1 = _slices(cntp)

    inv, Rx, Zx, Cx, hN1, AhN1 = _tpre(
        x, ax0, ax1, c0, c1,
        p["ssx"]["Wl"].T, p["ssx"]["Wr"].T, b(p["ssx"]["b"]),
        p["sux"]["Wl"].T, p["sux"]["Wr"].T, b(p["sux"]["b"]),
        p["scx"]["Wl"].T, p["scx"]["Wr"].T, b(p["scx"]["b"]),
        b(p["sh"]["b"]), b(p["bb1"]), b(p["bb2"]), p["Wb2"].T)

    q1, z1 = _t2_direct(
        Rx, Zx, hN1, AhN1,
        p["ssh"]["Wl"].T, p["ssh"]["Wr"].T, b(p["ssh"]["b"]),
        p["suh"]["Wl"].T, p["suh"]["Wr"].T, b(p["suh"]["b"]))
    aq1p = _agg(q1, src_p, dst_agg)
    aq10, aq11 = _slices(aq1p)
    h1 = _t3(Cx, q1, z1, hN1, aq10, aq11, inv,
             p["sch"]["Wl"].T, p["sch"]["Wr"].T, b(p["sch"]["b"]))

    ahp = _agg(h1, src_p, dst_agg)
    ah0, ah1 = _slices(ahp)
    hN2 = _t1(h1, ah0, ah1, inv,
              p["sh"]["Wl"].T, p["sh"]["Wr"].T, b(p["sh"]["b"]),
              p["Wb1"].T, b(p["bb1"]), p["Wb2"].T, b(p["bb2"]))
    ahnp = _agg(hN2, src_p, dst_agg)
    ahn0, ahn1 = _slices(ahnp)
    q2, z2 = _t2_comb(
        Rx, Zx, hN2, ahn0, ahn1, inv,
        p["ssh"]["Wl"].T, p["ssh"]["Wr"].T, b(p["ssh"]["b"]),
        p["suh"]["Wl"].T, p["suh"]["Wr"].T, b(p["suh"]["b"]))
    aq2p = _agg(q2, src_p, dst_agg)
    aq20, aq21 = _slices(aq2p)
    h2, hA, hB = _t3f(Cx, q2, z2, hN2, aq20, aq21, inv,
                      p["sch"]["Wl"].T, p["sch"]["Wr"].T, b(p["sch"]["b"]),
                      p["We1"][:, :_D].T, b(p["be1"]), p["We1"][:, _D:].T)

    p16 = _edge_mlp(hA, hB, src_edge, dst_edge, p["We2"][0])
    pred_pad = _efin(p16, jnp.reshape(p["be2"], (1, 1)))
    pred = pred_pad[:_E]
    return (pred, h2)


# speedup vs baseline: 4.0549x; 1.0058x over previous
import functools

import jax
import jax.numpy as jnp
from jax import lax
from jax.experimental import pallas as pl
from jax.experimental.pallas import tpu as pltpu
from jax.experimental.pallas import tpu_sc as plsc

_N = 10000
_D = 128
_NC = 2
_NS = 16
_NW = _NC * _NS
_E = 320000


def _chunking(c):
    g = -(-_E // (_NW * c))
    g += g % 2
    return g, _NW * g * c


_CA = 64
_GA, _EPA = _chunking(_CA)
_C = 128
_G, _EPAD = _chunking(_C)
_NROWS = _N + 112
_RPB = _NROWS // _NS

_mesh = plsc.VectorSubcoreMesh(core_axis_name="c", subcore_axis_name="s")


@functools.partial(
    pl.kernel, mesh=_mesh,
    out_type=jax.ShapeDtypeStruct((_NC * _NROWS, _D), jnp.float32),
    scratch_types=[
        pltpu.VMEM((_GA * _CA,), jnp.int32),
        pltpu.VMEM((_CA,), jnp.int32),
        pltpu.VMEM((_CA,), jnp.int32),
        pltpu.VMEM((_CA, _D), jnp.float32),
        pltpu.VMEM((_CA, _D), jnp.float32),
        pltpu.VMEM_SHARED((_NROWS, _D), jnp.float32),
        pltpu.SemaphoreType.DMA,
        pltpu.SemaphoreType.DMA,
        pltpu.SemaphoreType.DMA,
        pltpu.SemaphoreType.DMA,
    ])
def _agg(table, srcp, dstp, out, src_all, dst_v0, dst_v1, rows0, rows1,
         shacc, semG0, semG1, semD0, semD1):
    cid = lax.axis_index("c")
    sid = lax.axis_index("s")
    wid = sid * _NC + cid
    r0 = sid * _RPB
    base = wid * (_GA * _CA)

    pltpu.sync_copy(srcp.at[pl.ds(base, _GA * _CA)], src_all)

    def zrow(r, _):
        for c8 in range(_D // 16):
            rows0[r, pl.ds(c8 * 16, 16)] = jnp.zeros((16,), jnp.float32)
        return _
    lax.fori_loop(0, _CA, zrow, None)
    nz = -(-_RPB // _CA)

    def zcopy(i, _):
        off = jnp.minimum(r0 + i * _CA, r0 + _RPB - _CA)
        pltpu.sync_copy(rows0, shacc.at[pl.ds(off, _CA)])
        return _
    lax.fori_loop(0, nz, zcopy, None)

    def gather(g, rows, sem):
        return pltpu.make_async_copy(
            table.at[src_all.at[pl.ds(g * _CA, _CA)]], rows, sem)

    def dload(g, dst_v, sem):
        return pltpu.make_async_copy(
            dstp.at[pl.ds(base + g * _CA, _CA)], dst_v, sem)

    pltpu.async_copy(dstp.at[pl.ds(base, _CA)], dst_v0, semD0)
    pltpu.async_copy(table.at[src_all.at[pl.ds(0, _CA)]], rows0, semG0)
    plsc.subcore_barrier()

    def outer(g2, _):
        g0 = 2 * g2
        dload(g0 + 1, dst_v1, semD1).start()
        gather(g0 + 1, rows1, semG1).start()
        gather(g0, rows0, semG0).wait()
        dload(g0, dst_v0, semD0).wait()
        pltpu.sync_copy(rows0, shacc.at[dst_v0], add=True)

        @pl.when(g2 + 1 < _GA // 2)
        def _pref():
            dload(g0 + 2, dst_v0, semD0).start()
            gather(g0 + 2, rows0, semG0).start()
        gather(g0 + 1, rows1, semG1).wait()
        dload(g0 + 1, dst_v1, semD1).wait()
        pltpu.sync_copy(rows1, shacc.at[dst_v1], add=True)
        return _
    lax.fori_loop(0, _GA // 2, outer, None)

    plsc.subcore_barrier()
    pltpu.sync_copy(shacc.at[pl.ds(r0, _RPB)],
                    out.at[pl.ds(cid * _NROWS + r0, _RPB)])


@functools.partial(
    pl.kernel, mesh=_mesh,
    out_type=jax.ShapeDtypeStruct((_NC * _NROWS, _D), jnp.float32),
    scratch_types=[
        pltpu.VMEM((_CA,), jnp.int32),
        pltpu.VMEM((_CA,), jnp.int32),
        pltpu.VMEM((_CA, _D), jnp.float32),
        pltpu.VMEM_SHARED((_NROWS, _D), jnp.float32),
        pltpu.SemaphoreType.DMA,
        pltpu.SemaphoreType.DMA,
    ])
def _count(dstp, out, dst_v0, dst_v1, ones_v, cacc, semD0, semD1):
    cid = lax.axis_index("c")
    sid = lax.axis_index("s")
    wid = sid * _NC + cid
    r0 = sid * _RPB
    base = wid * (_GA * _CA)

    def zrow(r, _):
        for c8 in range(_D // 16):
            ones_v[r, pl.ds(c8 * 16, 16)] = jnp.zeros((16,), jnp.float32)
        return _
    lax.fori_loop(0, _CA, zrow, None)
    nz = -(-_RPB // _CA)

    def zcopy(i, _):
        off = jnp.minimum(r0 + i * _CA, r0 + _RPB - _CA)
        pltpu.sync_copy(ones_v, cacc.at[pl.ds(off, _CA)])
        return _
    lax.fori_loop(0, nz, zcopy, None)

    def orow(r, _):
        for c8 in range(_D // 16):
            ones_v[r, pl.ds(c8 * 16, 16)] = jnp.ones((16,), jnp.float32)
        return _
    lax.fori_loop(0, _CA, orow, None)

    def dload(g, dst_v, sem):
        return pltpu.make_async_copy(
            dstp.at[pl.ds(base + g * _CA, _CA)], dst_v, sem)

    pltpu.async_copy(dstp.at[pl.ds(base, _CA)], dst_v0, semD0)
    plsc.subcore_barrier()

    def outer(g2, _):
        g0 = 2 * g2
        dload(g0 + 1, dst_v1, semD1).start()
        dload(g0, dst_v0, semD0).wait()
        pltpu.sync_copy(ones_v, cacc.at[dst_v0], add=True)

        @pl.when(g2 + 1 < _GA // 2)
        def _pref():
            dload(g0 + 2, dst_v0, semD0).start()
        dload(g0 + 1, dst_v1, semD1).wait()
        pltpu.sync_copy(ones_v, cacc.at[dst_v1], add=True)
        return _
    lax.fori_loop(0, _GA // 2, outer, None)

    plsc.subcore_barrier()
    pltpu.sync_copy(cacc.at[pl.ds(r0, _RPB)],
                    out.at[pl.ds(cid * _NROWS + r0, _RPB)])


@functools.partial(
    pl.kernel, mesh=_mesh,
    out_type=jax.ShapeDtypeStruct((_EPAD, 16), jnp.float32),
    scratch_types=[
        pltpu.VMEM((_G, _C), jnp.int32),
        pltpu.VMEM((_G, _C), jnp.int32),
        pltpu.VMEM((_C, _D), jnp.float32),
        pltpu.VMEM((_C, _D), jnp.float32),
        pltpu.VMEM((_C, _D), jnp.float32),
        pltpu.VMEM((_C, _D), jnp.float32),
        pltpu.VMEM((_D,), jnp.float32),
        pltpu.VMEM((_C, 16), jnp.float32),
        pltpu.VMEM((_C, 16), jnp.float32),
        pltpu.SemaphoreType.DMA,
        pltpu.SemaphoreType.DMA,
        pltpu.SemaphoreType.DMA,
        pltpu.SemaphoreType.DMA,
        pltpu.SemaphoreType.DMA,
        pltpu.SemaphoreType.DMA,
    ])
def _edge_mlp(hA, hB, srcp3, dstp3, w2, out, src_all, dst_all,
              rowsA0, rowsB0, rowsA1, rowsB1, w2_v, pbuf0, pbuf1,
              semA0, semB0, semA1, semB1, semO0, semO1):
    cid = lax.axis_index("c")
    sid = lax.axis_index("s")
    wid = sid * _NC + cid
    base = wid * (_G * _C)

    pltpu.sync_copy(srcp3.at[wid], src_all)
    pltpu.sync_copy(dstp3.at[wid], dst_all)
    pltpu.sync_copy(w2, w2_v)
    wregs = [w2_v[pl.ds(k * 16, 16)] for k in range(_D // 16)]

    pltpu.async_copy(hA.at[src_all.at[0]], rowsA0, semA0)
    pltpu.async_copy(hB.at[dst_all.at[0]], rowsB0, semB0)

    def compute(rowsA, rowsB, pbuf):
        def edge(e4, _):
            e = e4 * 4
            accs = [jnp.zeros((16,), jnp.float32) for _ in range(4)]
            for k in range(_D // 16):
                for j in range(4):
                    va = rowsA[e + j, pl.ds(k * 16, 16)]
                    vb = rowsB[e + j, pl.ds(k * 16, 16)]
                    accs[j] = accs[j] + jnp.maximum(va + vb, 0.0) * wregs[k]
            for j in range(4):
                pbuf[e + j, pl.ds(0, 16)] = accs[j]
            return _
        lax.fori_loop(0, _C // 4, edge, None)

    def outer(g2, _):
        g0 = 2 * g2
        pltpu.async_copy(hA.at[src_all.at[g0 + 1]], rowsA1, semA1)
        pltpu.async_copy(hB.at[dst_all.at[g0 + 1]], rowsB1, semB1)
        pltpu.make_async_copy(hA.at[src_all.at[g0]], rowsA0, semA0).wait()
        pltpu.make_async_copy(hB.at[dst_all.at[g0]], rowsB0, semB0).wait()

        @pl.when(g2 > 0)
        def _drain0():
            pltpu.make_async_copy(
                pbuf0, out.at[pl.ds(base + (g0 - 2) * _C, _C)], semO0).wait()
        compute(rowsA0, rowsB0, pbuf0)
        pltpu.async_copy(pbuf0, out.at[pl.ds(base + g0 * _C, _C)], semO0)

        @pl.when(g2 + 1 < _G // 2)
        def _pref():
            pltpu.async_copy(hA.at[src_all.at[g0 + 2]], rowsA0, semA0)
            pltpu.async_copy(hB.at[dst_all.at[g0 + 2]], rowsB0, semB0)
        pltpu.make_async_copy(hA.at[src_all.at[g0 + 1]], rowsA1, semA1).wait()
        pltpu.make_async_copy(hB.at[dst_all.at[g0 + 1]], rowsB1, semB1).wait()

        @pl.when(g2 > 0)
        def _drain1():
            pltpu.make_async_copy(
                pbuf1, out.at[pl.ds(base + (g0 - 1) * _C, _C)], semO1).wait()
        compute(rowsA1, rowsB1, pbuf1)
        pltpu.async_copy(pbuf1, out.at[pl.ds(base + (g0 + 1) * _C, _C)], semO1)
        return _
    lax.fori_loop(0, _G // 2, outer, None)

    pltpu.make_async_copy(pbuf0, out.at[pl.ds(base + (_G - 2) * _C, _C)],
                          semO0).wait()
    pltpu.make_async_copy(pbuf1, out.at[pl.ds(base + (_G - 1) * _C, _C)],
                          semO1).wait()


_EBN = 4096
_ENB = _EPAD // _EBN


def _efin_body(p16, be2, out):
    s = jnp.sum(p16[...], axis=1, keepdims=True) + be2[...]
    out[...] = jax.nn.sigmoid(s)


_efin = pl.pallas_call(
    _efin_body,
    grid=(_ENB,),
    in_specs=[pl.BlockSpec((_EBN, 16), lambda i: (i, 0)),
              pl.BlockSpec((1, 1), lambda i: (0, 0))],
    out_specs=pl.BlockSpec((_EBN, 1), lambda i: (i, 0)),
    out_shape=jax.ShapeDtypeStruct((_EPAD, 1), jnp.float32),
)


_BN = 1000
_NB = _N // _BN


def _rows(bn=_BN, d=_D):
    return pl.BlockSpec((bn, d), lambda i: (i, 0))


def _wmat():
    return pl.BlockSpec((_D, _D), lambda i: (0, 0))


def _brow():
    return pl.BlockSpec((1, _D), lambda i: (0, 0))


def _f32(*shape):
    return jax.ShapeDtypeStruct(shape, jnp.float32)


def _dot(a, b):
    return jnp.dot(a, b, preferred_element_type=jnp.float32)


def _tpre_body(x, p0, p1, c0, c1, WLr, WRr, br, WLz, WRz, bz, WLc, WRc, bc,
               bsh, bb1, bb2, WB2,
               inv_o, rx_o, zx_o, cx_o, hn1_o, ahn1_o):
    cnt = c0[:, 0:1] + c1[:, 0:1]
    inv = 1.0 / jnp.maximum(cnt, 1.0)
    invb = jnp.broadcast_to(inv, (_BN, _D))
    inv_o[...] = invb
    A = (p0[...] + p1[...]) * invb
    xx = x[...]
    rx_o[...] = _dot(A, WLr[...]) + _dot(xx, WRr[...]) + br[...]
    zx_o[...] = _dot(A, WLz[...]) + _dot(xx, WRz[...]) + bz[...]
    cx_o[...] = _dot(A, WLc[...]) + _dot(xx, WRc[...]) + bc[...]
    beta = jnp.tanh(bb1[...] + _dot(bsh[...], WB2[...]) + bb2[...])
    hn1_o[...] = jnp.broadcast_to(beta, (_BN, _D))
    ahn1_o[...] = jnp.where(cnt > 0.0, 1.0, 0.0) * beta


_tpre = pl.pallas_call(
    _tpre_body,
    grid=(_NB,),
    in_specs=[_rows(), _rows(), _rows(), _rows(), _rows(),
              _wmat(), _wmat(), _brow(), _wmat(), _wmat(), _brow(),
              _wmat(), _wmat(), _brow(), _brow(), _brow(), _brow(), _wmat()],
    out_specs=[_rows()] * 6,
    out_shape=[_f32(_N, _D)] * 6,
)


def _t1_body(h, p0, p1, inv, WLs, WRs, bs, WB1, bb1, WB2, bb2, hn_o):
    A = (p0[...] + p1[...]) * inv[...]
    hh = h[...]
    hN0 = _dot(A, WLs[...]) + _dot(hh, WRs[...]) + bs[...]
    beta = jnp.tanh(_dot(hh, WB1[...]) + bb1[...] + _dot(hN0, WB2[...]) + bb2[...])
    hn_o[...] = hh + beta


_t1 = pl.pallas_call(
    _t1_body,
    grid=(_NB,),
    in_specs=[_rows()] * 4 + [_wmat(), _wmat(), _brow(), _wmat(), _brow(),
                              _wmat(), _brow()],
    out_specs=_rows(),
    out_shape=_f32(_N, _D),
)


def _t2_body(combine, *args):
    if combine:
        (rx, zx, hN, p0, p1, inv, WLr, WRr, br, WLz, WRz, bz, q_o, z_o) = args
        A = (p0[...] + p1[...]) * inv[...]
    else:
        (rx, zx, hN, Ai, WLr, WRr, br, WLz, WRz, bz, q_o, z_o) = args
        A = Ai[...]
    h = hN[...]
    r = jax.nn.sigmoid(rx[...] + _dot(A, WLr[...]) + _dot(h, WRr[...]) + br[...])
    z = jax.nn.sigmoid(zx[...] + _dot(A, WLz[...]) + _dot(h, WRz[...]) + bz[...])
    q_o[...] = r * h
    z_o[...] = z


_t2_direct = pl.pallas_call(
    functools.partial(_t2_body, False),
    grid=(_NB,),
    in_specs=[_rows()] * 4 + [_wmat(), _wmat(), _brow(), _wmat(), _wmat(),
                              _brow()],
    out_specs=[_rows()] * 2,
    out_shape=[_f32(_N, _D)] * 2,
)

_t2_comb = pl.pallas_call(
    functools.partial(_t2_body, True),
    grid=(_NB,),
    in_specs=[_rows()] * 6 + [_wmat(), _wmat(), _brow(), _wmat(), _wmat(),
                              _brow()],
    out_specs=[_rows()] * 2,
    out_shape=[_f32(_N, _D)] * 2,
)


def _t3_body(final, *args):
    if final:
        (cx, q, z, hN, p0, p1, inv, WLc, WRc, bc, WA, ba, WB,
         h_o, ha_o, hb_o) = args
    else:
        (cx, q, z, hN, p0, p1, inv, WLc, WRc, bc, h_o) = args
    A = (p0[...] + p1[...]) * inv[...]
    qq = q[...]
    ht = jnp.tanh(cx[...] + _dot(A, WLc[...]) + _dot(qq, WRc[...]) + bc[...])
    zz = z[...]
    h = (1.0 - zz) * hN[...] + zz * ht
    h_o[...] = h
    if final:
        ha_o[...] = _dot(h, WA[...]) + ba[...]
        hb_o[...] = _dot(h, WB[...])


_t3 = pl.pallas_call(
    functools.partial(_t3_body, False),
    grid=(_NB,),
    in_specs=[_rows()] * 7 + [_wmat(), _wmat(), _brow()],
    out_specs=_rows(),
    out_shape=_f32(_N, _D),
)

_t3f = pl.pallas_call(
    functools.partial(_t3_body, True),
    grid=(_NB,),
    in_specs=[_rows()] * 7 + [_wmat(), _wmat(), _brow(), _wmat(), _brow(),
                              _wmat()],
    out_specs=[_rows()] * 3,
    out_shape=[_f32(_N, _D)] * 3,
)


def _slices(part):
    return part[0:_N], part[_NROWS:_NROWS + _N]


def kernel(x, edge_index, params):
    p = params
    src = edge_index[0]
    dst = edge_index[1]
    pad_a = jnp.zeros((_EPA - _E,), jnp.int32)
    pad_e = jnp.zeros((_EPAD - _E,), jnp.int32)
    src_p = jnp.concatenate([src, pad_a])
    dst_agg = jnp.concatenate([dst, jnp.full((_EPA - _E,), _N, jnp.int32)])
    src_edge = jnp.concatenate([src, pad_e]).reshape(_NW, _G, _C)
    dst_edge = jnp.concatenate([dst, pad_e]).reshape(_NW, _G, _C)

    def b(v):
        return jnp.reshape(v, (1, _D))

    cntp = _count(dst_agg)
    axp = _agg(x, src_p, dst_agg)
    ax0, ax1 = _slices(axp)
    c0, c1 = _slices(cntp)

    inv, Rx, Zx, Cx, hN1, AhN1 = _tpre(
        x, ax0, ax1, c0, c1,
        p["ssx"]["Wl"].T, p["ssx"]["Wr"].T, b(p["ssx"]["b"]),
        p["sux"]["Wl"].T, p["sux"]["Wr"].T, b(p["sux"]["b"]),
        p["scx"]["Wl"].T, p["scx"]["Wr"].T, b(p["scx"]["b"]),
        b(p["sh"]["b"]), b(p["bb1"]), b(p["bb2"]), p["Wb2"].T)

    q1, z1 = _t2_direct(
        Rx, Zx, hN1, AhN1,
        p["ssh"]["Wl"].T, p["ssh"]["Wr"].T, b(p["ssh"]["b"]),
        p["suh"]["Wl"].T, p["suh"]["Wr"].T, b(p["suh"]["b"]))
    aq1p = _agg(q1, src_p, dst_agg)
    aq10, aq11 = _slices(aq1p)
    h1 = _t3(Cx, q1, z1, hN1, aq10, aq11, inv,
             p["sch"]["Wl"].T, p["sch"]["Wr"].T, b(p["sch"]["b"]))

    ahp = _agg(h1, src_p, dst_agg)
    ah0, ah1 = _slices(ahp)
    hN2 = _t1(h1, ah0, ah1, inv,
              p["sh"]["Wl"].T, p["sh"]["Wr"].T, b(p["sh"]["b"]),
              p["Wb1"].T, b(p["bb1"]), p["Wb2"].T, b(p["bb2"]))
    ahnp = _agg(hN2, src_p, dst_agg)
    ahn0, ahn1 = _slices(ahnp)
    q2, z2 = _t2_comb(
        Rx, Zx, hN2, ahn0, ahn1, inv,
        p["ssh"]["Wl"].T, p["ssh"]["Wr"].T, b(p["ssh"]["b"]),
        p["suh"]["Wl"].T, p["suh"]["Wr"].T, b(p["suh"]["b"]))
    aq2p = _agg(q2, src_p, dst_agg)
    aq20, aq21 = _slices(aq2p)
    h2, hA, hB = _t3f(Cx, q2, z2, hN2, aq20, aq21, inv,
                      p["sch"]["Wl"].T, p["sch"]["Wr"].T, b(p["sch"]["b"]),
                      p["We1"][:, :_D].T, b(p["be1"]), p["We1"][:, _D:].T)

    p16 = _edge_mlp(hA, hB, src_edge, dst_edge, p["We2"][0])
    pred_pad = _efin(p16, jnp.reshape(p["be2"], (1, 1)))
    pred = pred_pad[:_E]
    return (pred, h2)


# speedup vs baseline: 4.1964x; 1.0349x over previous
import functools

import jax
import jax.numpy as jnp
from jax import lax
from jax.experimental import pallas as pl
from jax.experimental.pallas import tpu as pltpu
from jax.experimental.pallas import tpu_sc as plsc

_N = 10000
_D = 128
_NC = 2
_NS = 16
_NW = _NC * _NS
_E = 320000


def _chunking(c):
    g = -(-_E // (_NW * c))
    g += g % 2
    return g, _NW * g * c


_CA = 64
_GA, _EPA = _chunking(_CA)
_C = 128
_G, _EPAD = _chunking(_C)
_NROWS = _N + 112
_RPB = _NROWS // _NS

_mesh = plsc.VectorSubcoreMesh(core_axis_name="c", subcore_axis_name="s")


@functools.partial(
    pl.kernel, mesh=_mesh,
    out_type=jax.ShapeDtypeStruct((_NC * _NROWS, _D), jnp.float32),
    scratch_types=[
        pltpu.VMEM((_GA * _CA,), jnp.int32),
        pltpu.VMEM((_CA,), jnp.int32),
        pltpu.VMEM((_CA,), jnp.int32),
        pltpu.VMEM((_CA, _D), jnp.float32),
        pltpu.VMEM((_CA, _D), jnp.float32),
        pltpu.VMEM_SHARED((_NROWS, _D), jnp.float32),
        pltpu.SemaphoreType.DMA,
        pltpu.SemaphoreType.DMA,
        pltpu.SemaphoreType.DMA,
        pltpu.SemaphoreType.DMA,
    ])
def _agg(table, srcp, dstp, out, src_all, dst_v0, dst_v1, rows0, rows1,
         shacc, semG0, semG1, semD0, semD1):
    cid = lax.axis_index("c")
    sid = lax.axis_index("s")
    wid = sid * _NC + cid
    r0 = sid * _RPB
    base = wid * (_GA * _CA)

    pltpu.sync_copy(srcp.at[pl.ds(base, _GA * _CA)], src_all)

    def zrow(r, _):
        for c8 in range(_D // 16):
            rows0[r, pl.ds(c8 * 16, 16)] = jnp.zeros((16,), jnp.float32)
        return _
    lax.fori_loop(0, _CA, zrow, None)
    nz = -(-_RPB // _CA)

    def zcopy(i, _):
        off = jnp.minimum(r0 + i * _CA, r0 + _RPB - _CA)
        pltpu.sync_copy(rows0, shacc.at[pl.ds(off, _CA)])
        return _
    lax.fori_loop(0, nz, zcopy, None)

    def gather(g, rows, sem):
        return pltpu.make_async_copy(
            table.at[src_all.at[pl.ds(g * _CA, _CA)]], rows, sem)

    def dload(g, dst_v, sem):
        return pltpu.make_async_copy(
            dstp.at[pl.ds(base + g * _CA, _CA)], dst_v, sem)

    pltpu.async_copy(dstp.at[pl.ds(base, _CA)], dst_v0, semD0)
    pltpu.async_copy(table.at[src_all.at[pl.ds(0, _CA)]], rows0, semG0)
    plsc.subcore_barrier()

    def outer(g2, _):
        g0 = 2 * g2
        dload(g0 + 1, dst_v1, semD1).start()
        gather(g0 + 1, rows1, semG1).start()
        gather(g0, rows0, semG0).wait()
        dload(g0, dst_v0, semD0).wait()
        pltpu.sync_copy(rows0, shacc.at[dst_v0], add=True)

        @pl.when(g2 + 1 < _GA // 2)
        def _pref():
            dload(g0 + 2, dst_v0, semD0).start()
            gather(g0 + 2, rows0, semG0).start()
        gather(g0 + 1, rows1, semG1).wait()
        dload(g0 + 1, dst_v1, semD1).wait()
        pltpu.sync_copy(rows1, shacc.at[dst_v1], add=True)
        return _
    lax.fori_loop(0, _GA // 2, outer, None)

    plsc.subcore_barrier()
    pltpu.sync_copy(shacc.at[pl.ds(r0, _RPB)],
                    out.at[pl.ds(cid * _NROWS + r0, _RPB)])


@functools.partial(
    pl.kernel, mesh=_mesh,
    out_type=jax.ShapeDtypeStruct((_NC * _NROWS, _D), jnp.float32),
    scratch_types=[
        pltpu.VMEM((_CA,), jnp.int32),
        pltpu.VMEM((_CA,), jnp.int32),
        pltpu.VMEM((_CA, _D), jnp.float32),
        pltpu.VMEM_SHARED((_NROWS, _D), jnp.float32),
        pltpu.SemaphoreType.DMA,
        pltpu.SemaphoreType.DMA,
    ])
def _count(dstp, out, dst_v0, dst_v1, ones_v, cacc, semD0, semD1):
    cid = lax.axis_index("c")
    sid = lax.axis_index("s")
    wid = sid * _NC + cid
    r0 = sid * _RPB
    base = wid * (_GA * _CA)

    def zrow(r, _):
        for c8 in range(_D // 16):
            ones_v[r, pl.ds(c8 * 16, 16)] = jnp.zeros((16,), jnp.float32)
        return _
    lax.fori_loop(0, _CA, zrow, None)
    nz = -(-_RPB // _CA)

    def zcopy(i, _):
        off = jnp.minimum(r0 + i * _CA, r0 + _RPB - _CA)
        pltpu.sync_copy(ones_v, cacc.at[pl.ds(off, _CA)])
        return _
    lax.fori_loop(0, nz, zcopy, None)

    def orow(r, _):
        for c8 in range(_D // 16):
            ones_v[r, pl.ds(c8 * 16, 16)] = jnp.ones((16,), jnp.float32)
        return _
    lax.fori_loop(0, _CA, orow, None)

    def dload(g, dst_v, sem):
        return pltpu.make_async_copy(
            dstp.at[pl.ds(base + g * _CA, _CA)], dst_v, sem)

    pltpu.async_copy(dstp.at[pl.ds(base, _CA)], dst_v0, semD0)
    plsc.subcore_barrier()

    def outer(g2, _):
        g0 = 2 * g2
        dload(g0 + 1, dst_v1, semD1).start()
        dload(g0, dst_v0, semD0).wait()
        pltpu.sync_copy(ones_v, cacc.at[dst_v0], add=True)

        @pl.when(g2 + 1 < _GA // 2)
        def _pref():
            dload(g0 + 2, dst_v0, semD0).start()
        dload(g0 + 1, dst_v1, semD1).wait()
        pltpu.sync_copy(ones_v, cacc.at[dst_v1], add=True)
        return _
    lax.fori_loop(0, _GA // 2, outer, None)

    plsc.subcore_barrier()
    pltpu.sync_copy(cacc.at[pl.ds(r0, _RPB)],
                    out.at[pl.ds(cid * _NROWS + r0, _RPB)])


@functools.partial(
    pl.kernel, mesh=_mesh,
    out_type=jax.ShapeDtypeStruct((_EPAD, 16), jnp.float32),
    scratch_types=[
        pltpu.VMEM((_G, _C), jnp.int32),
        pltpu.VMEM((_G, _C), jnp.int32),
        pltpu.VMEM((_C, _D), jnp.float32),
        pltpu.VMEM((_C, _D), jnp.float32),
        pltpu.VMEM((_C, _D), jnp.float32),
        pltpu.VMEM((_C, _D), jnp.float32),
        pltpu.VMEM((_D,), jnp.float32),
        pltpu.VMEM((_C, 16), jnp.float32),
        pltpu.VMEM((_C, 16), jnp.float32),
        pltpu.SemaphoreType.DMA,
        pltpu.SemaphoreType.DMA,
        pltpu.SemaphoreType.DMA,
        pltpu.SemaphoreType.DMA,
        pltpu.SemaphoreType.DMA,
        pltpu.SemaphoreType.DMA,
    ])
def _edge_mlp(hA, hB, srcp3, dstp3, w2, out, src_all, dst_all,
              rowsA0, rowsB0, rowsA1, rowsB1, w2_v, pbuf0, pbuf1,
              semA0, semB0, semA1, semB1, semO0, semO1):
    cid = lax.axis_index("c")
    sid = lax.axis_index("s")
    wid = sid * _NC + cid
    base = wid * (_G * _C)

    pltpu.sync_copy(srcp3.at[wid], src_all)
    pltpu.sync_copy(dstp3.at[wid], dst_all)
    pltpu.sync_copy(w2, w2_v)
    wregs = [w2_v[pl.ds(k * 16, 16)] for k in range(_D // 16)]

    pltpu.async_copy(hA.at[src_all.at[0]], rowsA0, semA0)
    pltpu.async_copy(hB.at[dst_all.at[0]], rowsB0, semB0)

    def compute(rowsA, rowsB, pbuf):
        def edge(e4, _):
            e = e4 * 4
            accs = [jnp.zeros((16,), jnp.float32) for _ in range(4)]
            for k in range(_D // 16):
                for j in range(4):
                    va = rowsA[e + j, pl.ds(k * 16, 16)]
                    vb = rowsB[e + j, pl.ds(k * 16, 16)]
                    accs[j] = accs[j] + jnp.maximum(va + vb, 0.0) * wregs[k]
            for j in range(4):
                pbuf[e + j, pl.ds(0, 16)] = accs[j]
            return _
        lax.fori_loop(0, _C // 4, edge, None)

    def outer(g2, _):
        g0 = 2 * g2
        pltpu.async_copy(hA.at[src_all.at[g0 + 1]], rowsA1, semA1)
        pltpu.async_copy(hB.at[dst_all.at[g0 + 1]], rowsB1, semB1)
        pltpu.make_async_copy(hA.at[src_all.at[g0]], rowsA0, semA0).wait()
        pltpu.make_async_copy(hB.at[dst_all.at[g0]], rowsB0, semB0).wait()

        @pl.when(g2 > 0)
        def _drain0():
            pltpu.make_async_copy(
                pbuf0, out.at[pl.ds(base + (g0 - 2) * _C, _C)], semO0).wait()
        compute(rowsA0, rowsB0, pbuf0)
        pltpu.async_copy(pbuf0, out.at[pl.ds(base + g0 * _C, _C)], semO0)

        @pl.when(g2 + 1 < _G // 2)
        def _pref():
            pltpu.async_copy(hA.at[src_all.at[g0 + 2]], rowsA0, semA0)
            pltpu.async_copy(hB.at[dst_all.at[g0 + 2]], rowsB0, semB0)
        pltpu.make_async_copy(hA.at[src_all.at[g0 + 1]], rowsA1, semA1).wait()
        pltpu.make_async_copy(hB.at[dst_all.at[g0 + 1]], rowsB1, semB1).wait()

        @pl.when(g2 > 0)
        def _drain1():
            pltpu.make_async_copy(
                pbuf1, out.at[pl.ds(base + (g0 - 1) * _C, _C)], semO1).wait()
        compute(rowsA1, rowsB1, pbuf1)
        pltpu.async_copy(pbuf1, out.at[pl.ds(base + (g0 + 1) * _C, _C)], semO1)
        return _
    lax.fori_loop(0, _G // 2, outer, None)

    pltpu.make_async_copy(pbuf0, out.at[pl.ds(base + (_G - 2) * _C, _C)],
                          semO0).wait()
    pltpu.make_async_copy(pbuf1, out.at[pl.ds(base + (_G - 1) * _C, _C)],
                          semO1).wait()


_ENB = 10
_EBN = _EPAD // _ENB


def _efin_body(p16, be2, out):
    s = jnp.sum(p16[...], axis=1) + be2[0, 0]
    out[...] = jax.nn.sigmoid(s)


_efin = pl.pallas_call(
    _efin_body,
    grid=(_ENB,),
    in_specs=[pl.BlockSpec((_EBN, 16), lambda i: (i, 0)),
              pl.BlockSpec(memory_space=pltpu.SMEM)],
    out_specs=pl.BlockSpec((_EBN,), lambda i: (i,)),
    out_shape=jax.ShapeDtypeStruct((_EPAD,), jnp.float32),
)


_BN = 1000
_NB = _N // _BN


def _rows(bn=_BN, d=_D):
    return pl.BlockSpec((bn, d), lambda i: (i, 0))


def _wmat():
    return pl.BlockSpec((_D, _D), lambda i: (0, 0))


def _brow():
    return pl.BlockSpec((1, _D), lambda i: (0, 0))


def _f32(*shape):
    return jax.ShapeDtypeStruct(shape, jnp.float32)


def _dot(a, b):
    return jnp.dot(a, b, preferred_element_type=jnp.float32)


def _tpre_body(x, p0, p1, c0, c1, WLr, WRr, br, WLz, WRz, bz, WLc, WRc, bc,
               bsh, bb1, bb2, WB2, hWLr, hWRr, hbr, hWLz, hWRz, hbz,
               inv_o, rx_o, zx_o, cx_o, hn1_o, q_o, z_o):
    cnt = c0[:, 0:1] + c1[:, 0:1]
    inv = 1.0 / jnp.maximum(cnt, 1.0)
    invb = jnp.broadcast_to(inv, (_BN, _D))
    inv_o[...] = invb
    A = (p0[...] + p1[...]) * invb
    xx = x[...]
    rx = _dot(A, WLr[...]) + _dot(xx, WRr[...]) + br[...]
    zx = _dot(A, WLz[...]) + _dot(xx, WRz[...]) + bz[...]
    rx_o[...] = rx
    zx_o[...] = zx
    cx_o[...] = _dot(A, WLc[...]) + _dot(xx, WRc[...]) + bc[...]
    beta = jnp.tanh(bb1[...] + _dot(bsh[...], WB2[...]) + bb2[...])
    hn1 = jnp.broadcast_to(beta, (_BN, _D))
    hn1_o[...] = hn1
    ahn1 = jnp.where(cnt > 0.0, 1.0, 0.0) * beta
    r = jax.nn.sigmoid(rx + _dot(ahn1, hWLr[...]) + _dot(hn1, hWRr[...])
                       + hbr[...])
    z = jax.nn.sigmoid(zx + _dot(ahn1, hWLz[...]) + _dot(hn1, hWRz[...])
                       + hbz[...])
    q_o[...] = r * hn1
    z_o[...] = z


_tpre = pl.pallas_call(
    _tpre_body,
    grid=(_NB,),
    in_specs=[_rows(), _rows(), _rows(), _rows(), _rows(),
              _wmat(), _wmat(), _brow(), _wmat(), _wmat(), _brow(),
              _wmat(), _wmat(), _brow(), _brow(), _brow(), _brow(), _wmat(),
              _wmat(), _wmat(), _brow(), _wmat(), _wmat(), _brow()],
    out_specs=[_rows()] * 7,
    out_shape=[_f32(_N, _D)] * 7,
)


def _t1_body(h, p0, p1, inv, WLs, WRs, bs, WB1, bb1, WB2, bb2, hn_o):
    A = (p0[...] + p1[...]) * inv[...]
    hh = h[...]
    hN0 = _dot(A, WLs[...]) + _dot(hh, WRs[...]) + bs[...]
    beta = jnp.tanh(_dot(hh, WB1[...]) + bb1[...] + _dot(hN0, WB2[...]) + bb2[...])
    hn_o[...] = hh + beta


_t1 = pl.pallas_call(
    _t1_body,
    grid=(_NB,),
    in_specs=[_rows()] * 4 + [_wmat(), _wmat(), _brow(), _wmat(), _brow(),
                              _wmat(), _brow()],
    out_specs=_rows(),
    out_shape=_f32(_N, _D),
)


def _t2_body(rx, zx, hN, p0, p1, inv, WLr, WRr, br, WLz, WRz, bz, q_o, z_o):
    A = (p0[...] + p1[...]) * inv[...]
    h = hN[...]
    r = jax.nn.sigmoid(rx[...] + _dot(A, WLr[...]) + _dot(h, WRr[...]) + br[...])
    z = jax.nn.sigmoid(zx[...] + _dot(A, WLz[...]) + _dot(h, WRz[...]) + bz[...])
    q_o[...] = r * h
    z_o[...] = z


_t2_comb = pl.pallas_call(
    _t2_body,
    grid=(_NB,),
    in_specs=[_rows()] * 6 + [_wmat(), _wmat(), _brow(), _wmat(), _wmat(),
                              _brow()],
    out_specs=[_rows()] * 2,
    out_shape=[_f32(_N, _D)] * 2,
)


def _t3_body(final, *args):
    if final:
        (cx, q, z, hN, p0, p1, inv, WLc, WRc, bc, WA, ba, WB,
         h_o, ha_o, hb_o) = args
    else:
        (cx, q, z, hN, p0, p1, inv, WLc, WRc, bc, h_o) = args
    A = (p0[...] + p1[...]) * inv[...]
    qq = q[...]
    ht = jnp.tanh(cx[...] + _dot(A, WLc[...]) + _dot(qq, WRc[...]) + bc[...])
    zz = z[...]
    h = (1.0 - zz) * hN[...] + zz * ht
    h_o[...] = h
    if final:
        ha_o[...] = _dot(h, WA[...]) + ba[...]
        hb_o[...] = _dot(h, WB[...])


_t3 = pl.pallas_call(
    functools.partial(_t3_body, False),
    grid=(_NB,),
    in_specs=[_rows()] * 7 + [_wmat(), _wmat(), _brow()],
    out_specs=_rows(),
    out_shape=_f32(_N, _D),
)

_t3f = pl.pallas_call(
    functools.partial(_t3_body, True),
    grid=(_NB,),
    in_specs=[_rows()] * 7 + [_wmat(), _wmat(), _brow(), _wmat(), _brow(),
                              _wmat()],
    out_specs=[_rows()] * 3,
    out_shape=[_f32(_N, _D)] * 3,
)


def _slices(part):
    return part[0:_N], part[_NROWS:_NROWS + _N]


def kernel(x, edge_index, params):
    p = params
    src = edge_index[0]
    dst = edge_index[1]
    pad_a = jnp.zeros((_EPA - _E,), jnp.int32)
    pad_e = jnp.zeros((_EPAD - _E,), jnp.int32)
    src_p = jnp.concatenate([src, pad_a])
    dst_agg = jnp.concatenate([dst, jnp.full((_EPA - _E,), _N, jnp.int32)])
    src_edge = jnp.concatenate([src, pad_e]).reshape(_NW, _G, _C)
    dst_edge = jnp.concatenate([dst, pad_e]).reshape(_NW, _G, _C)

    def b(v):
        return jnp.reshape(v, (1, _D))

    cntp = _count(dst_agg)
    axp = _agg(x, src_p, dst_agg)
    ax0, ax1 = _slices(axp)
    c0, c1 = _slices(cntp)

    inv, Rx, Zx, Cx, hN1, q1, z1 = _tpre(
        x, ax0, ax1, c0, c1,
        p["ssx"]["Wl"].T, p["ssx"]["Wr"].T, b(p["ssx"]["b"]),
        p["sux"]["Wl"].T, p["sux"]["Wr"].T, b(p["sux"]["b"]),
        p["scx"]["Wl"].T, p["scx"]["Wr"].T, b(p["scx"]["b"]),
        b(p["sh"]["b"]), b(p["bb1"]), b(p["bb2"]), p["Wb2"].T,
        p["ssh"]["Wl"].T, p["ssh"]["Wr"].T, b(p["ssh"]["b"]),
        p["suh"]["Wl"].T, p["suh"]["Wr"].T, b(p["suh"]["b"]))
    aq1p = _agg(q1, src_p, dst_agg)
    aq10, aq11 = _slices(aq1p)
    h1 = _t3(Cx, q1, z1, hN1, aq10, aq11, inv,
             p["sch"]["Wl"].T, p["sch"]["Wr"].T, b(p["sch"]["b"]))

    ahp = _agg(h1, src_p, dst_agg)
    ah0, ah1 = _slices(ahp)
    hN2 = _t1(h1, ah0, ah1, inv,
              p["sh"]["Wl"].T, p["sh"]["Wr"].T, b(p["sh"]["b"]),
              p["Wb1"].T, b(p["bb1"]), p["Wb2"].T, b(p["bb2"]))
    ahnp = _agg(hN2, src_p, dst_agg)
    ahn0, ahn1 = _slices(ahnp)
    q2, z2 = _t2_comb(
        Rx, Zx, hN2, ahn0, ahn1, inv,
        p["ssh"]["Wl"].T, p["ssh"]["Wr"].T, b(p["ssh"]["b"]),
        p["suh"]["Wl"].T, p["suh"]["Wr"].T, b(p["suh"]["b"]))
    aq2p = _agg(q2, src_p, dst_agg)
    aq20, aq21 = _slices(aq2p)
    h2, hA, hB = _t3f(Cx, q2, z2, hN2, aq20, aq21, inv,
                      p["sch"]["Wl"].T, p["sch"]["Wr"].T, b(p["sch"]["b"]),
                      p["We1"][:, :_D].T, b(p["be1"]), p["We1"][:, _D:].T)

    p16 = _edge_mlp(hA, hB, src_edge, dst_edge, p["We2"][0])
    pred_pad = _efin(p16, jnp.reshape(p["be2"], (1, 1)))
    pred = jnp.reshape(pred_pad[:_E], (_E, 1))
    return (pred, h2)


# speedup vs baseline: 4.4957x; 1.0713x over previous
import functools

import jax
import jax.numpy as jnp
from jax import lax
from jax.experimental import pallas as pl
from jax.experimental.pallas import tpu as pltpu
from jax.experimental.pallas import tpu_sc as plsc

_N = 10000
_D = 128
_NC = 2
_NS = 16
_NW = _NC * _NS
_E = 320000


def _chunking(c):
    g = -(-_E // (_NW * c))
    g += g % 2
    return g, _NW * g * c


_CA = 64
_GA, _EPA = _chunking(_CA)
_C = 128
_G, _EPAD = _chunking(_C)
_NROWS = _N + 112
_RPB = _NROWS // _NS

_mesh = plsc.VectorSubcoreMesh(core_axis_name="c", subcore_axis_name="s")


@functools.partial(
    pl.kernel, mesh=_mesh,
    out_type=jax.ShapeDtypeStruct((_NC * _NROWS, _D), jnp.float32),
    scratch_types=[
        pltpu.VMEM((_GA * _CA,), jnp.int32),
        pltpu.VMEM((_CA,), jnp.int32),
        pltpu.VMEM((_CA,), jnp.int32),
        pltpu.VMEM((_CA, _D), jnp.float32),
        pltpu.VMEM((_CA, _D), jnp.float32),
        pltpu.VMEM_SHARED((_NROWS, _D), jnp.float32),
        pltpu.SemaphoreType.DMA,
        pltpu.SemaphoreType.DMA,
        pltpu.SemaphoreType.DMA,
        pltpu.SemaphoreType.DMA,
    ])
def _agg(table, srcp, dstp, out, src_all, dst_v0, dst_v1, rows0, rows1,
         shacc, semG0, semG1, semD0, semD1):
    cid = lax.axis_index("c")
    sid = lax.axis_index("s")
    wid = sid * _NC + cid
    r0 = sid * _RPB
    base = wid * (_GA * _CA)

    pltpu.sync_copy(srcp.at[pl.ds(base, _GA * _CA)], src_all)

    def zrow(r, _):
        for c8 in range(_D // 16):
            rows0[r, pl.ds(c8 * 16, 16)] = jnp.zeros((16,), jnp.float32)
        return _
    lax.fori_loop(0, _CA, zrow, None)
    nz = -(-_RPB // _CA)

    def zcopy(i, _):
        off = jnp.minimum(r0 + i * _CA, r0 + _RPB - _CA)
        pltpu.sync_copy(rows0, shacc.at[pl.ds(off, _CA)])
        return _
    lax.fori_loop(0, nz, zcopy, None)

    def gather(g, rows, sem):
        return pltpu.make_async_copy(
            table.at[src_all.at[pl.ds(g * _CA, _CA)]], rows, sem)

    def dload(g, dst_v, sem):
        return pltpu.make_async_copy(
            dstp.at[pl.ds(base + g * _CA, _CA)], dst_v, sem)

    pltpu.async_copy(dstp.at[pl.ds(base, _CA)], dst_v0, semD0)
    pltpu.async_copy(table.at[src_all.at[pl.ds(0, _CA)]], rows0, semG0)
    plsc.subcore_barrier()

    def outer(g2, _):
        g0 = 2 * g2
        dload(g0 + 1, dst_v1, semD1).start()
        gather(g0 + 1, rows1, semG1).start()
        gather(g0, rows0, semG0).wait()
        dload(g0, dst_v0, semD0).wait()
        pltpu.sync_copy(rows0, shacc.at[dst_v0], add=True)

        @pl.when(g2 + 1 < _GA // 2)
        def _pref():
            dload(g0 + 2, dst_v0, semD0).start()
            gather(g0 + 2, rows0, semG0).start()
        gather(g0 + 1, rows1, semG1).wait()
        dload(g0 + 1, dst_v1, semD1).wait()
        pltpu.sync_copy(rows1, shacc.at[dst_v1], add=True)
        return _
    lax.fori_loop(0, _GA // 2, outer, None)

    plsc.subcore_barrier()
    pltpu.sync_copy(shacc.at[pl.ds(r0, _RPB)],
                    out.at[pl.ds(cid * _NROWS + r0, _RPB)])


@functools.partial(
    pl.kernel, mesh=_mesh,
    out_type=jax.ShapeDtypeStruct((_NC * _NROWS, _D), jnp.float32),
    scratch_types=[
        pltpu.VMEM((_CA,), jnp.int32),
        pltpu.VMEM((_CA,), jnp.int32),
        pltpu.VMEM((_CA, _D), jnp.float32),
        pltpu.VMEM_SHARED((_NROWS, _D), jnp.float32),
        pltpu.SemaphoreType.DMA,
        pltpu.SemaphoreType.DMA,
    ])
def _count(dstp, out, dst_v0, dst_v1, ones_v, cacc, semD0, semD1):
    cid = lax.axis_index("c")
    sid = lax.axis_index("s")
    wid = sid * _NC + cid
    r0 = sid * _RPB
    base = wid * (_GA * _CA)

    def zrow(r, _):
        for c8 in range(_D // 16):
            ones_v[r, pl.ds(c8 * 16, 16)] = jnp.zeros((16,), jnp.float32)
        return _
    lax.fori_loop(0, _CA, zrow, None)
    nz = -(-_RPB // _CA)

    def zcopy(i, _):
        off = jnp.minimum(r0 + i * _CA, r0 + _RPB - _CA)
        pltpu.sync_copy(ones_v, cacc.at[pl.ds(off, _CA)])
        return _
    lax.fori_loop(0, nz, zcopy, None)

    def orow(r, _):
        for c8 in range(_D // 16):
            ones_v[r, pl.ds(c8 * 16, 16)] = jnp.ones((16,), jnp.float32)
        return _
    lax.fori_loop(0, _CA, orow, None)

    def dload(g, dst_v, sem):
        return pltpu.make_async_copy(
            dstp.at[pl.ds(base + g * _CA, _CA)], dst_v, sem)

    pltpu.async_copy(dstp.at[pl.ds(base, _CA)], dst_v0, semD0)
    plsc.subcore_barrier()

    def outer(g2, _):
        g0 = 2 * g2
        dload(g0 + 1, dst_v1, semD1).start()
        dload(g0, dst_v0, semD0).wait()
        pltpu.sync_copy(ones_v, cacc.at[dst_v0], add=True)

        @pl.when(g2 + 1 < _GA // 2)
        def _pref():
            dload(g0 + 2, dst_v0, semD0).start()
        dload(g0 + 1, dst_v1, semD1).wait()
        pltpu.sync_copy(ones_v, cacc.at[dst_v1], add=True)
        return _
    lax.fori_loop(0, _GA // 2, outer, None)

    plsc.subcore_barrier()
    pltpu.sync_copy(cacc.at[pl.ds(r0, _RPB)],
                    out.at[pl.ds(cid * _NROWS + r0, _RPB)])


_CR = _C // 8


@functools.partial(
    pl.kernel, mesh=_mesh,
    out_type=jax.ShapeDtypeStruct((_EPAD // 8, _D), jnp.float32),
    scratch_types=[
        pltpu.VMEM((_G, _C), jnp.int32),
        pltpu.VMEM((_G, _C), jnp.int32),
        pltpu.VMEM((_C, _D), jnp.float32),
        pltpu.VMEM((_C, _D), jnp.float32),
        pltpu.VMEM((_C, _D), jnp.float32),
        pltpu.VMEM((_C, _D), jnp.float32),
        pltpu.VMEM((_D,), jnp.float32),
        pltpu.VMEM((_CR, _D), jnp.float32),
        pltpu.VMEM((_CR, _D), jnp.float32),
        pltpu.SemaphoreType.DMA,
        pltpu.SemaphoreType.DMA,
        pltpu.SemaphoreType.DMA,
        pltpu.SemaphoreType.DMA,
        pltpu.SemaphoreType.DMA,
        pltpu.SemaphoreType.DMA,
    ])
def _edge_mlp(hA, hB, srcp3, dstp3, w2, out, src_all, dst_all,
              rowsA0, rowsB0, rowsA1, rowsB1, w2_v, pbuf0, pbuf1,
              semA0, semB0, semA1, semB1, semO0, semO1):
    cid = lax.axis_index("c")
    sid = lax.axis_index("s")
    wid = sid * _NC + cid
    base = wid * (_G * _CR)

    pltpu.sync_copy(srcp3.at[wid], src_all)
    pltpu.sync_copy(dstp3.at[wid], dst_all)
    pltpu.sync_copy(w2, w2_v)
    wregs = [w2_v[pl.ds(k * 16, 16)] for k in range(_D // 16)]

    pltpu.async_copy(hA.at[src_all.at[0]], rowsA0, semA0)
    pltpu.async_copy(hB.at[dst_all.at[0]], rowsB0, semB0)

    def compute(rowsA, rowsB, pbuf):
        def edge(e8, _):
            e = e8 * 8
            accs = [jnp.zeros((16,), jnp.float32) for _ in range(8)]
            for k in range(_D // 16):
                for j in range(8):
                    va = rowsA[e + j, pl.ds(k * 16, 16)]
                    vb = rowsB[e + j, pl.ds(k * 16, 16)]
                    accs[j] = accs[j] + jnp.maximum(va + vb, 0.0) * wregs[k]
            for j in range(8):
                pbuf[e8, pl.ds(j * 16, 16)] = accs[j]
            return _
        lax.fori_loop(0, _C // 8, edge, None)

    def outer(g2, _):
        g0 = 2 * g2
        pltpu.async_copy(hA.at[src_all.at[g0 + 1]], rowsA1, semA1)
        pltpu.async_copy(hB.at[dst_all.at[g0 + 1]], rowsB1, semB1)
        pltpu.make_async_copy(hA.at[src_all.at[g0]], rowsA0, semA0).wait()
        pltpu.make_async_copy(hB.at[dst_all.at[g0]], rowsB0, semB0).wait()

        @pl.when(g2 > 0)
        def _drain0():
            pltpu.make_async_copy(
                pbuf0, out.at[pl.ds(base + (g0 - 2) * _CR, _CR)], semO0).wait()
        compute(rowsA0, rowsB0, pbuf0)
        pltpu.async_copy(pbuf0, out.at[pl.ds(base + g0 * _CR, _CR)], semO0)

        @pl.when(g2 + 1 < _G // 2)
        def _pref():
            pltpu.async_copy(hA.at[src_all.at[g0 + 2]], rowsA0, semA0)
            pltpu.async_copy(hB.at[dst_all.at[g0 + 2]], rowsB0, semB0)
        pltpu.make_async_copy(hA.at[src_all.at[g0 + 1]], rowsA1, semA1).wait()
        pltpu.make_async_copy(hB.at[dst_all.at[g0 + 1]], rowsB1, semB1).wait()

        @pl.when(g2 > 0)
        def _drain1():
            pltpu.make_async_copy(
                pbuf1, out.at[pl.ds(base + (g0 - 1) * _CR, _CR)], semO1).wait()
        compute(rowsA1, rowsB1, pbuf1)
        pltpu.async_copy(pbuf1, out.at[pl.ds(base + (g0 + 1) * _CR, _CR)],
                         semO1)
        return _
    lax.fori_loop(0, _G // 2, outer, None)

    pltpu.make_async_copy(pbuf0, out.at[pl.ds(base + (_G - 2) * _CR, _CR)],
                          semO0).wait()
    pltpu.make_async_copy(pbuf1, out.at[pl.ds(base + (_G - 1) * _CR, _CR)],
                          semO1).wait()


_ENB = 10
_EBN = _EPAD // 8 // _ENB


def _efin_body(p8, m, be2, out):
    s = _dot(p8[...], m[...]) + be2[0, 0]
    out[...] = jax.nn.sigmoid(s)


_efin = pl.pallas_call(
    _efin_body,
    grid=(_ENB,),
    in_specs=[pl.BlockSpec((_EBN, _D), lambda i: (i, 0)),
              pl.BlockSpec((_D, 8), lambda i: (0, 0)),
              pl.BlockSpec(memory_space=pltpu.SMEM)],
    out_specs=pl.BlockSpec((_EBN, 8), lambda i: (i, 0)),
    out_shape=jax.ShapeDtypeStruct((_EPAD // 8, 8), jnp.float32),
)


_BN = 1000
_NB = _N // _BN


def _rows(bn=_BN, d=_D):
    return pl.BlockSpec((bn, d), lambda i: (i, 0))


def _wmat():
    return pl.BlockSpec((_D, _D), lambda i: (0, 0))


def _brow():
    return pl.BlockSpec((1, _D), lambda i: (0, 0))


def _f32(*shape):
    return jax.ShapeDtypeStruct(shape, jnp.float32)


def _dot(a, b):
    return jnp.dot(a, b, preferred_element_type=jnp.float32)


def _tpre_body(x, p0, p1, c0, c1, WLr, WRr, br, WLz, WRz, bz, WLc, WRc, bc,
               bsh, bb1, bb2, WB2, hWLr, hWRr, hbr, hWLz, hWRz, hbz,
               inv_o, rx_o, zx_o, cx_o, hn1_o, q_o, z_o):
    cnt = c0[:, 0:1] + c1[:, 0:1]
    inv = 1.0 / jnp.maximum(cnt, 1.0)
    invb = jnp.broadcast_to(inv, (_BN, _D))
    inv_o[...] = invb
    A = (p0[...] + p1[...]) * invb
    xx = x[...]
    rx = _dot(A, WLr[...]) + _dot(xx, WRr[...]) + br[...]
    zx = _dot(A, WLz[...]) + _dot(xx, WRz[...]) + bz[...]
    rx_o[...] = rx
    zx_o[...] = zx
    cx_o[...] = _dot(A, WLc[...]) + _dot(xx, WRc[...]) + bc[...]
    beta = jnp.tanh(bb1[...] + _dot(bsh[...], WB2[...]) + bb2[...])
    hn1 = jnp.broadcast_to(beta, (_BN, _D))
    hn1_o[...] = hn1
    ahn1 = jnp.where(cnt > 0.0, 1.0, 0.0) * beta
    r = jax.nn.sigmoid(rx + _dot(ahn1, hWLr[...]) + _dot(hn1, hWRr[...])
                       + hbr[...])
    z = jax.nn.sigmoid(zx + _dot(ahn1, hWLz[...]) + _dot(hn1, hWRz[...])
                       + hbz[...])
    q_o[...] = r * hn1
    z_o[...] = z


_tpre = pl.pallas_call(
    _tpre_body,
    grid=(_NB,),
    in_specs=[_rows(), _rows(), _rows(), _rows(), _rows(),
              _wmat(), _wmat(), _brow(), _wmat(), _wmat(), _brow(),
              _wmat(), _wmat(), _brow(), _brow(), _brow(), _brow(), _wmat(),
              _wmat(), _wmat(), _brow(), _wmat(), _wmat(), _brow()],
    out_specs=[_rows()] * 7,
    out_shape=[_f32(_N, _D)] * 7,
)


def _t1_body(h, p0, p1, inv, WLs, WRs, bs, WB1, bb1, WB2, bb2, hn_o):
    A = (p0[...] + p1[...]) * inv[...]
    hh = h[...]
    hN0 = _dot(A, WLs[...]) + _dot(hh, WRs[...]) + bs[...]
    beta = jnp.tanh(_dot(hh, WB1[...]) + bb1[...] + _dot(hN0, WB2[...]) + bb2[...])
    hn_o[...] = hh + beta


_t1 = pl.pallas_call(
    _t1_body,
    grid=(_NB,),
    in_specs=[_rows()] * 4 + [_wmat(), _wmat(), _brow(), _wmat(), _brow(),
                              _wmat(), _brow()],
    out_specs=_rows(),
    out_shape=_f32(_N, _D),
)


def _t2_body(rx, zx, hN, p0, p1, inv, WLr, WRr, br, WLz, WRz, bz, q_o, z_o):
    A = (p0[...] + p1[...]) * inv[...]
    h = hN[...]
    r = jax.nn.sigmoid(rx[...] + _dot(A, WLr[...]) + _dot(h, WRr[...]) + br[...])
    z = jax.nn.sigmoid(zx[...] + _dot(A, WLz[...]) + _dot(h, WRz[...]) + bz[...])
    q_o[...] = r * h
    z_o[...] = z


_t2_comb = pl.pallas_call(
    _t2_body,
    grid=(_NB,),
    in_specs=[_rows()] * 6 + [_wmat(), _wmat(), _brow(), _wmat(), _wmat(),
                              _brow()],
    out_specs=[_rows()] * 2,
    out_shape=[_f32(_N, _D)] * 2,
)


def _t3_body(final, *args):
    if final:
        (cx, q, z, hN, p0, p1, inv, WLc, WRc, bc, WA, ba, WB,
         h_o, ha_o, hb_o) = args
    else:
        (cx, q, z, hN, p0, p1, inv, WLc, WRc, bc, h_o) = args
    A = (p0[...] + p1[...]) * inv[...]
    qq = q[...]
    ht = jnp.tanh(cx[...] + _dot(A, WLc[...]) + _dot(qq, WRc[...]) + bc[...])
    zz = z[...]
    h = (1.0 - zz) * hN[...] + zz * ht
    h_o[...] = h
    if final:
        ha_o[...] = _dot(h, WA[...]) + ba[...]
        hb_o[...] = _dot(h, WB[...])


_t3 = pl.pallas_call(
    functools.partial(_t3_body, False),
    grid=(_NB,),
    in_specs=[_rows()] * 7 + [_wmat(), _wmat(), _brow()],
    out_specs=_rows(),
    out_shape=_f32(_N, _D),
)

_t3f = pl.pallas_call(
    functools.partial(_t3_body, True),
    grid=(_NB,),
    in_specs=[_rows()] * 7 + [_wmat(), _wmat(), _brow(), _wmat(), _brow(),
                              _wmat()],
    out_specs=[_rows()] * 3,
    out_shape=[_f32(_N, _D)] * 3,
)


def _slices(part):
    return part[0:_N], part[_NROWS:_NROWS + _N]


def kernel(x, edge_index, params):
    p = params
    src = edge_index[0]
    dst = edge_index[1]
    pad_a = jnp.zeros((_EPA - _E,), jnp.int32)
    pad_e = jnp.zeros((_EPAD - _E,), jnp.int32)
    src_p = jnp.concatenate([src, pad_a])
    dst_agg = jnp.concatenate([dst, jnp.full((_EPA - _E,), _N, jnp.int32)])
    src_edge = jnp.concatenate([src, pad_e]).reshape(_NW, _G, _C)
    dst_edge = jnp.concatenate([dst, pad_e]).reshape(_NW, _G, _C)

    def b(v):
        return jnp.reshape(v, (1, _D))

    cntp = _count(dst_agg)
    axp = _agg(x, src_p, dst_agg)
    ax0, ax1 = _slices(axp)
    c0, c1 = _slices(cntp)

    inv, Rx, Zx, Cx, hN1, q1, z1 = _tpre(
        x, ax0, ax1, c0, c1,
        p["ssx"]["Wl"].T, p["ssx"]["Wr"].T, b(p["ssx"]["b"]),
        p["sux"]["Wl"].T, p["sux"]["Wr"].T, b(p["sux"]["b"]),
        p["scx"]["Wl"].T, p["scx"]["Wr"].T, b(p["scx"]["b"]),
        b(p["sh"]["b"]), b(p["bb1"]), b(p["bb2"]), p["Wb2"].T,
        p["ssh"]["Wl"].T, p["ssh"]["Wr"].T, b(p["ssh"]["b"]),
        p["suh"]["Wl"].T, p["suh"]["Wr"].T, b(p["suh"]["b"]))
    aq1p = _agg(q1, src_p, dst_agg)
    aq10, aq11 = _slices(aq1p)
    h1 = _t3(Cx, q1, z1, hN1, aq10, aq11, inv,
             p["sch"]["Wl"].T, p["sch"]["Wr"].T, b(p["sch"]["b"]))

    ahp = _agg(h1, src_p, dst_agg)
    ah0, ah1 = _slices(ahp)
    hN2 = _t1(h1, ah0, ah1, inv,
              p["sh"]["Wl"].T, p["sh"]["Wr"].T, b(p["sh"]["b"]),
              p["Wb1"].T, b(p["bb1"]), p["Wb2"].T, b(p["bb2"]))
    ahnp = _agg(hN2, src_p, dst_agg)
    ahn0, ahn1 = _slices(ahnp)
    q2, z2 = _t2_comb(
        Rx, Zx, hN2, ahn0, ahn1, inv,
        p["ssh"]["Wl"].T, p["ssh"]["Wr"].T, b(p["ssh"]["b"]),
        p["suh"]["Wl"].T, p["suh"]["Wr"].T, b(p["suh"]["b"]))
    aq2p = _agg(q2, src_p, dst_agg)
    aq20, aq21 = _slices(aq2p)
    h2, hA, hB = _t3f(Cx, q2, z2, hN2, aq20, aq21, inv,
                      p["sch"]["Wl"].T, p["sch"]["Wr"].T, b(p["sch"]["b"]),
                      p["We1"][:, :_D].T, b(p["be1"]), p["We1"][:, _D:].T)

    p8 = _edge_mlp(hA, hB, src_edge, dst_edge, p["We2"][0])
    m = jnp.kron(jnp.eye(8, dtype=jnp.float32),
                 jnp.ones((16, 1), jnp.float32))
    pred8 = _efin(p8, m, jnp.reshape(p["be2"], (1, 1)))
    pred = jnp.reshape(jnp.reshape(pred8, (_EPAD,))[:_E], (_E, 1))
    return (pred, h2)


# speedup vs baseline: 4.5073x; 1.0026x over previous
import functools

import jax
import jax.numpy as jnp
from jax import lax
from jax.experimental import pallas as pl
from jax.experimental.pallas import tpu as pltpu
from jax.experimental.pallas import tpu_sc as plsc

_N = 10000
_D = 128
_NC = 2
_NS = 16
_NW = _NC * _NS
_E = 320000


def _chunking(c):
    g = -(-_E // (_NW * c))
    g += g % 2
    return g, _NW * g * c


_CA = 64
_GA, _EPA = _chunking(_CA)
_C = 128
_G, _EPAD = _chunking(_C)
_NROWS = _N + 112
_RPB = _NROWS // _NS

_mesh = plsc.VectorSubcoreMesh(core_axis_name="c", subcore_axis_name="s")


@functools.partial(
    pl.kernel, mesh=_mesh,
    out_type=jax.ShapeDtypeStruct((_NC * _NROWS, _D), jnp.float32),
    scratch_types=[
        pltpu.VMEM((_GA * _CA,), jnp.int32),
        pltpu.VMEM((_GA * _CA,), jnp.int32),
        pltpu.VMEM((_CA, _D), jnp.float32),
        pltpu.VMEM((_CA, _D), jnp.float32),
        pltpu.VMEM_SHARED((_NROWS, _D), jnp.float32),
        pltpu.SemaphoreType.DMA,
        pltpu.SemaphoreType.DMA,
    ])
def _agg(table, srcp, dstp, out, src_all, dst_all, rows0, rows1,
         shacc, semG0, semG1):
    cid = lax.axis_index("c")
    sid = lax.axis_index("s")
    wid = sid * _NC + cid
    r0 = sid * _RPB
    base = wid * (_GA * _CA)

    pltpu.sync_copy(srcp.at[pl.ds(base, _GA * _CA)], src_all)
    pltpu.sync_copy(dstp.at[pl.ds(base, _GA * _CA)], dst_all)

    def zrow(r, _):
        for c8 in range(_D // 16):
            rows0[r, pl.ds(c8 * 16, 16)] = jnp.zeros((16,), jnp.float32)
        return _
    lax.fori_loop(0, _CA, zrow, None)
    nz = -(-_RPB // _CA)

    def zcopy(i, _):
        off = jnp.minimum(r0 + i * _CA, r0 + _RPB - _CA)
        pltpu.sync_copy(rows0, shacc.at[pl.ds(off, _CA)])
        return _
    lax.fori_loop(0, nz, zcopy, None)

    def gather(g, rows, sem):
        return pltpu.make_async_copy(
            table.at[src_all.at[pl.ds(g * _CA, _CA)]], rows, sem)

    def dix(g):
        return dst_all.at[pl.ds(g * _CA, _CA)]

    pltpu.async_copy(table.at[src_all.at[pl.ds(0, _CA)]], rows0, semG0)
    plsc.subcore_barrier()

    def outer(g2, _):
        g0 = 2 * g2
        gather(g0 + 1, rows1, semG1).start()
        gather(g0, rows0, semG0).wait()
        pltpu.sync_copy(rows0, shacc.at[dix(g0)], add=True)

        @pl.when(g2 + 1 < _GA // 2)
        def _pref():
            gather(g0 + 2, rows0, semG0).start()
        gather(g0 + 1, rows1, semG1).wait()
        pltpu.sync_copy(rows1, shacc.at[dix(g0 + 1)], add=True)
        return _
    lax.fori_loop(0, _GA // 2, outer, None)

    plsc.subcore_barrier()
    pltpu.sync_copy(shacc.at[pl.ds(r0, _RPB)],
                    out.at[pl.ds(cid * _NROWS + r0, _RPB)])


@functools.partial(
    pl.kernel, mesh=_mesh,
    out_type=jax.ShapeDtypeStruct((_NC * _NROWS, _D), jnp.float32),
    scratch_types=[
        pltpu.VMEM((_GA * _CA,), jnp.int32),
        pltpu.VMEM((_CA, _D), jnp.float32),
        pltpu.VMEM_SHARED((_NROWS, _D), jnp.float32),
    ])
def _count(dstp, out, dst_all, ones_v, cacc):
    cid = lax.axis_index("c")
    sid = lax.axis_index("s")
    wid = sid * _NC + cid
    r0 = sid * _RPB
    base = wid * (_GA * _CA)
    pltpu.sync_copy(dstp.at[pl.ds(base, _GA * _CA)], dst_all)

    def zrow(r, _):
        for c8 in range(_D // 16):
            ones_v[r, pl.ds(c8 * 16, 16)] = jnp.zeros((16,), jnp.float32)
        return _
    lax.fori_loop(0, _CA, zrow, None)
    nz = -(-_RPB // _CA)

    def zcopy(i, _):
        off = jnp.minimum(r0 + i * _CA, r0 + _RPB - _CA)
        pltpu.sync_copy(ones_v, cacc.at[pl.ds(off, _CA)])
        return _
    lax.fori_loop(0, nz, zcopy, None)

    def orow(r, _):
        for c8 in range(_D // 16):
            ones_v[r, pl.ds(c8 * 16, 16)] = jnp.ones((16,), jnp.float32)
        return _
    lax.fori_loop(0, _CA, orow, None)

    plsc.subcore_barrier()

    def outer(g, _):
        pltpu.sync_copy(ones_v, cacc.at[dst_all.at[pl.ds(g * _CA, _CA)]],
                        add=True)
        return _
    lax.fori_loop(0, _GA, outer, None)

    plsc.subcore_barrier()
    pltpu.sync_copy(cacc.at[pl.ds(r0, _RPB)],
                    out.at[pl.ds(cid * _NROWS + r0, _RPB)])


_CR = _C // 8


@functools.partial(
    pl.kernel, mesh=_mesh,
    out_type=jax.ShapeDtypeStruct((_EPAD // 8, _D), jnp.float32),
    scratch_types=[
        pltpu.VMEM((_G, _C), jnp.int32),
        pltpu.VMEM((_G, _C), jnp.int32),
        pltpu.VMEM((_C, _D), jnp.float32),
        pltpu.VMEM((_C, _D), jnp.float32),
        pltpu.VMEM((_C, _D), jnp.float32),
        pltpu.VMEM((_C, _D), jnp.float32),
        pltpu.VMEM((_D,), jnp.float32),
        pltpu.VMEM((_CR, _D), jnp.float32),
        pltpu.VMEM((_CR, _D), jnp.float32),
        pltpu.SemaphoreType.DMA,
        pltpu.SemaphoreType.DMA,
        pltpu.SemaphoreType.DMA,
        pltpu.SemaphoreType.DMA,
        pltpu.SemaphoreType.DMA,
        pltpu.SemaphoreType.DMA,
    ])
def _edge_mlp(hA, hB, srcp3, dstp3, w2, out, src_all, dst_all,
              rowsA0, rowsB0, rowsA1, rowsB1, w2_v, pbuf0, pbuf1,
              semA0, semB0, semA1, semB1, semO0, semO1):
    cid = lax.axis_index("c")
    sid = lax.axis_index("s")
    wid = sid * _NC + cid
    base = wid * (_G * _CR)

    pltpu.sync_copy(srcp3.at[wid], src_all)
    pltpu.sync_copy(dstp3.at[wid], dst_all)
    pltpu.sync_copy(w2, w2_v)
    wregs = [w2_v[pl.ds(k * 16, 16)] for k in range(_D // 16)]

    pltpu.async_copy(hA.at[src_all.at[0]], rowsA0, semA0)
    pltpu.async_copy(hB.at[dst_all.at[0]], rowsB0, semB0)

    def compute(rowsA, rowsB, pbuf):
        def edge(e8, _):
            e = e8 * 8
            accs = [jnp.zeros((16,), jnp.float32) for _ in range(8)]
            for k in range(_D // 16):
                for j in range(8):
                    va = rowsA[e + j, pl.ds(k * 16, 16)]
                    vb = rowsB[e + j, pl.ds(k * 16, 16)]
                    accs[j] = accs[j] + jnp.maximum(va + vb, 0.0) * wregs[k]
            for j in range(8):
                pbuf[e8, pl.ds(j * 16, 16)] = accs[j]
            return _
        lax.fori_loop(0, _C // 8, edge, None)

    def outer(g2, _):
        g0 = 2 * g2
        pltpu.async_copy(hA.at[src_all.at[g0 + 1]], rowsA1, semA1)
        pltpu.async_copy(hB.at[dst_all.at[g0 + 1]], rowsB1, semB1)
        pltpu.make_async_copy(hA.at[src_all.at[g0]], rowsA0, semA0).wait()
        pltpu.make_async_copy(hB.at[dst_all.at[g0]], rowsB0, semB0).wait()

        @pl.when(g2 > 0)
        def _drain0():
            pltpu.make_async_copy(
                pbuf0, out.at[pl.ds(base + (g0 - 2) * _CR, _CR)], semO0).wait()
        compute(rowsA0, rowsB0, pbuf0)
        pltpu.async_copy(pbuf0, out.at[pl.ds(base + g0 * _CR, _CR)], semO0)

        @pl.when(g2 + 1 < _G // 2)
        def _pref():
            pltpu.async_copy(hA.at[src_all.at[g0 + 2]], rowsA0, semA0)
            pltpu.async_copy(hB.at[dst_all.at[g0 + 2]], rowsB0, semB0)
        pltpu.make_async_copy(hA.at[src_all.at[g0 + 1]], rowsA1, semA1).wait()
        pltpu.make_async_copy(hB.at[dst_all.at[g0 + 1]], rowsB1, semB1).wait()

        @pl.when(g2 > 0)
        def _drain1():
            pltpu.make_async_copy(
                pbuf1, out.at[pl.ds(base + (g0 - 1) * _CR, _CR)], semO1).wait()
        compute(rowsA1, rowsB1, pbuf1)
        pltpu.async_copy(pbuf1, out.at[pl.ds(base + (g0 + 1) * _CR, _CR)],
                         semO1)
        return _
    lax.fori_loop(0, _G // 2, outer, None)

    pltpu.make_async_copy(pbuf0, out.at[pl.ds(base + (_G - 2) * _CR, _CR)],
                          semO0).wait()
    pltpu.make_async_copy(pbuf1, out.at[pl.ds(base + (_G - 1) * _CR, _CR)],
                          semO1).wait()


_ENB = 10
_EBN = _EPAD // 8 // _ENB


def _efin_body(p8, m, be2, out):
    s = _dot(p8[...], m[...]) + be2[0, 0]
    out[...] = jax.nn.sigmoid(s)


_efin = pl.pallas_call(
    _efin_body,
    grid=(_ENB,),
    in_specs=[pl.BlockSpec((_EBN, _D), lambda i: (i, 0)),
              pl.BlockSpec((_D, 8), lambda i: (0, 0)),
              pl.BlockSpec(memory_space=pltpu.SMEM)],
    out_specs=pl.BlockSpec((_EBN, 8), lambda i: (i, 0)),
    out_shape=jax.ShapeDtypeStruct((_EPAD // 8, 8), jnp.float32),
)


_BN = 1000
_NB = _N // _BN


def _rows(bn=_BN, d=_D):
    return pl.BlockSpec((bn, d), lambda i: (i, 0))


def _wmat():
    return pl.BlockSpec((_D, _D), lambda i: (0, 0))


def _brow():
    return pl.BlockSpec((1, _D), lambda i: (0, 0))


def _f32(*shape):
    return jax.ShapeDtypeStruct(shape, jnp.float32)


def _dot(a, b):
    return jnp.dot(a, b, preferred_element_type=jnp.float32)


def _tpre_body(x, p0, p1, c0, c1, WLr, WRr, br, WLz, WRz, bz, WLc, WRc, bc,
               bsh, bb1, bb2, WB2, hWLr, hWRr, hbr, hWLz, hWRz, hbz,
               inv_o, rx_o, zx_o, cx_o, hn1_o, q_o, z_o):
    cnt = c0[:, 0:1] + c1[:, 0:1]
    inv = 1.0 / jnp.maximum(cnt, 1.0)
    invb = jnp.broadcast_to(inv, (_BN, _D))
    inv_o[...] = invb
    A = (p0[...] + p1[...]) * invb
    xx = x[...]
    rx = _dot(A, WLr[...]) + _dot(xx, WRr[...]) + br[...]
    zx = _dot(A, WLz[...]) + _dot(xx, WRz[...]) + bz[...]
    rx_o[...] = rx
    zx_o[...] = zx
    cx_o[...] = _dot(A, WLc[...]) + _dot(xx, WRc[...]) + bc[...]
    beta = jnp.tanh(bb1[...] + _dot(bsh[...], WB2[...]) + bb2[...])
    hn1 = jnp.broadcast_to(beta, (_BN, _D))
    hn1_o[...] = hn1
    ahn1 = jnp.where(cnt > 0.0, 1.0, 0.0) * beta
    r = jax.nn.sigmoid(rx + _dot(ahn1, hWLr[...]) + _dot(hn1, hWRr[...])
                       + hbr[...])
    z = jax.nn.sigmoid(zx + _dot(ahn1, hWLz[...]) + _dot(hn1, hWRz[...])
                       + hbz[...])
    q_o[...] = r * hn1
    z_o[...] = z


_tpre = pl.pallas_call(
    _tpre_body,
    grid=(_NB,),
    in_specs=[_rows(), _rows(), _rows(), _rows(), _rows(),
              _wmat(), _wmat(), _brow(), _wmat(), _wmat(), _brow(),
              _wmat(), _wmat(), _brow(), _brow(), _brow(), _brow(), _wmat(),
              _wmat(), _wmat(), _brow(), _wmat(), _wmat(), _brow()],
    out_specs=[_rows()] * 7,
    out_shape=[_f32(_N, _D)] * 7,
)


def _t1_body(h, p0, p1, inv, WLs, WRs, bs, WB1, bb1, WB2, bb2, hn_o):
    A = (p0[...] + p1[...]) * inv[...]
    hh = h[...]
    hN0 = _dot(A, WLs[...]) + _dot(hh, WRs[...]) + bs[...]
    beta = jnp.tanh(_dot(hh, WB1[...]) + bb1[...] + _dot(hN0, WB2[...]) + bb2[...])
    hn_o[...] = hh + beta


_t1 = pl.pallas_call(
    _t1_body,
    grid=(_NB,),
    in_specs=[_rows()] * 4 + [_wmat(), _wmat(), _brow(), _wmat(), _brow(),
                              _wmat(), _brow()],
    out_specs=_rows(),
    out_shape=_f32(_N, _D),
)


def _t2_body(rx, zx, hN, p0, p1, inv, WLr, WRr, br, WLz, WRz, bz, q_o, z_o):
    A = (p0[...] + p1[...]) * inv[...]
    h = hN[...]
    r = jax.nn.sigmoid(rx[...] + _dot(A, WLr[...]) + _dot(h, WRr[...]) + br[...])
    z = jax.nn.sigmoid(zx[...] + _dot(A, WLz[...]) + _dot(h, WRz[...]) + bz[...])
    q_o[...] = r * h
    z_o[...] = z


_t2_comb = pl.pallas_call(
    _t2_body,
    grid=(_NB,),
    in_specs=[_rows()] * 6 + [_wmat(), _wmat(), _brow(), _wmat(), _wmat(),
                              _brow()],
    out_specs=[_rows()] * 2,
    out_shape=[_f32(_N, _D)] * 2,
)


def _t3_body(final, *args):
    if final:
        (cx, q, z, hN, p0, p1, inv, WLc, WRc, bc, WA, ba, WB,
         h_o, ha_o, hb_o) = args
    else:
        (cx, q, z, hN, p0, p1, inv, WLc, WRc, bc, h_o) = args
    A = (p0[...] + p1[...]) * inv[...]
    qq = q[...]
    ht = jnp.tanh(cx[...] + _dot(A, WLc[...]) + _dot(qq, WRc[...]) + bc[...])
    zz = z[...]
    h = (1.0 - zz) * hN[...] + zz * ht
    h_o[...] = h
    if final:
        ha_o[...] = _dot(h, WA[...]) + ba[...]
        hb_o[...] = _dot(h, WB[...])


_t3 = pl.pallas_call(
    functools.partial(_t3_body, False),
    grid=(_NB,),
    in_specs=[_rows()] * 7 + [_wmat(), _wmat(), _brow()],
    out_specs=_rows(),
    out_shape=_f32(_N, _D),
)

_t3f = pl.pallas_call(
    functools.partial(_t3_body, True),
    grid=(_NB,),
    in_specs=[_rows()] * 7 + [_wmat(), _wmat(), _brow(), _wmat(), _brow(),
                              _wmat()],
    out_specs=[_rows()] * 3,
    out_shape=[_f32(_N, _D)] * 3,
)


def _slices(part):
    return part[0:_N], part[_NROWS:_NROWS + _N]


def kernel(x, edge_index, params):
    p = params
    src = edge_index[0]
    dst = edge_index[1]
    pad_a = jnp.zeros((_EPA - _E,), jnp.int32)
    pad_e = jnp.zeros((_EPAD - _E,), jnp.int32)
    src_p = jnp.concatenate([src, pad_a])
    dst_agg = jnp.concatenate([dst, jnp.full((_EPA - _E,), _N, jnp.int32)])
    src_edge = jnp.concatenate([src, pad_e]).reshape(_NW, _G, _C)
    dst_edge = jnp.concatenate([dst, pad_e]).reshape(_NW, _G, _C)

    def b(v):
        return jnp.reshape(v, (1, _D))

    cntp = _count(dst_agg)
    axp = _agg(x, src_p, dst_agg)
    ax0, ax1 = _slices(axp)
    c0, c1 = _slices(cntp)

    inv, Rx, Zx, Cx, hN1, q1, z1 = _tpre(
        x, ax0, ax1, c0, c1,
        p["ssx"]["Wl"].T, p["ssx"]["Wr"].T, b(p["ssx"]["b"]),
        p["sux"]["Wl"].T, p["sux"]["Wr"].T, b(p["sux"]["b"]),
        p["scx"]["Wl"].T, p["scx"]["Wr"].T, b(p["scx"]["b"]),
        b(p["sh"]["b"]), b(p["bb1"]), b(p["bb2"]), p["Wb2"].T,
        p["ssh"]["Wl"].T, p["ssh"]["Wr"].T, b(p["ssh"]["b"]),
        p["suh"]["Wl"].T, p["suh"]["Wr"].T, b(p["suh"]["b"]))
    aq1p = _agg(q1, src_p, dst_agg)
    aq10, aq11 = _slices(aq1p)
    h1 = _t3(Cx, q1, z1, hN1, aq10, aq11, inv,
             p["sch"]["Wl"].T, p["sch"]["Wr"].T, b(p["sch"]["b"]))

    ahp = _agg(h1, src_p, dst_agg)
    ah0, ah1 = _slices(ahp)
    hN2 = _t1(h1, ah0, ah1, inv,
              p["sh"]["Wl"].T, p["sh"]["Wr"].T, b(p["sh"]["b"]),
              p["Wb1"].T, b(p["bb1"]), p["Wb2"].T, b(p["bb2"]))
    ahnp = _agg(hN2, src_p, dst_agg)
    ahn0, ahn1 = _slices(ahnp)
    q2, z2 = _t2_comb(
        Rx, Zx, hN2, ahn0, ahn1, inv,
        p["ssh"]["Wl"].T, p["ssh"]["Wr"].T, b(p["ssh"]["b"]),
        p["suh"]["Wl"].T, p["suh"]["Wr"].T, b(p["suh"]["b"]))
    aq2p = _agg(q2, src_p, dst_agg)
    aq20, aq21 = _slices(aq2p)
    h2, hA, hB = _t3f(Cx, q2, z2, hN2, aq20, aq21, inv,
                      p["sch"]["Wl"].T, p["sch"]["Wr"].T, b(p["sch"]["b"]),
                      p["We1"][:, :_D].T, b(p["be1"]), p["We1"][:, _D:].T)

    p8 = _edge_mlp(hA, hB, src_edge, dst_edge, p["We2"][0])
    m = jnp.kron(jnp.eye(8, dtype=jnp.float32),
                 jnp.ones((16, 1), jnp.float32))
    pred8 = _efin(p8, m, jnp.reshape(p["be2"], (1, 1)))
    pred = jnp.reshape(jnp.reshape(pred8, (_EPAD,))[:_E], (_E, 1))
    return (pred, h2)
